# baseline (speedup 1.0000x reference)
import functools

import jax
import jax.numpy as jnp
from jax import lax
from jax.experimental import pallas as pl
from jax.experimental.pallas import tpu as pltpu
from jax.experimental.pallas import tpu_sc as plsc

_SA1_SPECS = [(0.05, 16, [9, 16, 16, 32]), (0.1, 32, [9, 32, 32, 64])]
_SA2_SPECS = [(0.1, 16, [99, 64, 64, 128]), (0.2, 32, [99, 64, 96, 128])]
_NPOINT1, _NPOINT2 = 1024, 256


def _cumsum_lanes(x, n):
    s = 1
    while s < n:
        x = x + jnp.pad(x, ((0, 0), (s, 0)))[:, :n]
        s *= 2
    return x


def _fps_kernel(xs_ref, ys_ref, zs_ref, cx_ref, cy_ref, cz_ref, dist_ref,
                *, npoint, n):
    b = xs_ref.shape[0]
    xs = xs_ref[...]
    ys = ys_ref[...]
    zs = zs_ref[...]
    col = jax.lax.broadcasted_iota(jnp.int32, (b, n), 1)
    colp = jax.lax.broadcasted_iota(jnp.int32, (b, npoint), 1)
    dist_ref[...] = jnp.full((b, n), 1e10, jnp.float32)
    cx_ref[...] = jnp.zeros((b, npoint), jnp.float32)
    cy_ref[...] = jnp.zeros((b, npoint), jnp.float32)
    cz_ref[...] = jnp.zeros((b, npoint), jnp.float32)

    def body(t, far):
        sel = col == far
        cx = jnp.sum(jnp.where(sel, xs, 0.0), axis=1, keepdims=True)
        cy = jnp.sum(jnp.where(sel, ys, 0.0), axis=1, keepdims=True)
        cz = jnp.sum(jnp.where(sel, zs, 0.0), axis=1, keepdims=True)
        hit = colp == t
        cx_ref[...] = jnp.where(hit, cx, cx_ref[...])
        cy_ref[...] = jnp.where(hit, cy, cy_ref[...])
        cz_ref[...] = jnp.where(hit, cz, cz_ref[...])
        dx = xs - cx
        dy = ys - cy
        dz = zs - cz
        d = dx * dx + dy * dy + dz * dz
        dist = jnp.minimum(dist_ref[...], d)
        dist_ref[...] = dist
        mx = jnp.max(dist, axis=1, keepdims=True)
        far_new = jnp.min(jnp.where(dist == mx, col, n), axis=1, keepdims=True)
        return far_new

    jax.lax.fori_loop(0, npoint, body, jnp.zeros((b, 1), jnp.int32))


def _fps_centers(xs, ys, zs, npoint):
    b, n = xs.shape
    out_shape = [jax.ShapeDtypeStruct((b, npoint), jnp.float32)] * 3
    return pl.pallas_call(
        functools.partial(_fps_kernel, npoint=npoint, n=n),
        out_shape=out_shape,
        scratch_shapes=[pltpu.VMEM((b, n), jnp.float32)],
    )(xs, ys, zs)


def _group_kernel(xs_ref, ys_ref, zs_ref, pts_ref, cx_ref, cy_ref, cz_ref,
                  out_ref, *, r2, ns, rb, n, c, chunk):
    xs = xs_ref[0]
    ys = ys_ref[0]
    zs = zs_ref[0]
    cxb = cx_ref[0]
    cyb = cy_ref[0]
    czb = cz_ref[0]
    dx = cxb - xs
    dy = cyb - ys
    dz = czb - zs
    sqr = dx * dx + dy * dy + dz * dz
    mask = sqr <= r2
    rank = _cumsum_lanes(mask.astype(jnp.int32), n)
    count = rank[:, n - 1:n].reshape(rb, 1, 1)
    kio = jax.lax.broadcasted_iota(jnp.int32, (rb, ns, 1), 1)
    tgt = jnp.where(kio < count, kio + 1, 1)
    rankm3 = jnp.where(mask, rank, 0).reshape(rb, 1, n)
    acc = jnp.zeros((rb * ns, c), jnp.float32)
    for j0 in range(0, n, chunk):
        sel = rankm3[:, :, j0:j0 + chunk] == tgt
        self_f = sel.astype(jnp.float32).reshape(rb * ns, chunk)
        acc = acc + jnp.dot(self_f, pts_ref[0, j0:j0 + chunk, :],
                            preferred_element_type=jnp.float32,
                            precision=jax.lax.Precision.HIGHEST)
    acc3 = acc.reshape(rb, ns, c)
    chio = jax.lax.broadcasted_iota(jnp.int32, (rb, ns, c), 2)
    ctr = (jnp.where(chio == 0, cxb.reshape(rb, 1, 1), 0.0)
           + jnp.where(chio == 1, cyb.reshape(rb, 1, 1), 0.0)
           + jnp.where(chio == 2, czb.reshape(rb, 1, 1), 0.0))
    out_ref[0] = (acc3 - ctr).reshape(rb * ns, c)


def _group(xs3, ys3, zs3, pts, cx3, cy3, cz3, radius, ns, rb, chunk):
    b, _, n = xs3.shape
    c = pts.shape[-1]
    s = cx3.shape[1]
    r2 = radius * radius
    kern = functools.partial(_group_kernel, r2=r2, ns=ns, rb=rb, n=n, c=c,
                             chunk=chunk)
    row_spec = pl.BlockSpec((1, 1, n), lambda bi, si: (bi, 0, 0))
    ctr_spec = pl.BlockSpec((1, rb, 1), lambda bi, si: (bi, si, 0))
    return pl.pallas_call(
        kern,
        grid=(b, s // rb),
        in_specs=[row_spec, row_spec, row_spec,
                  pl.BlockSpec((1, n, c), lambda bi, si: (bi, 0, 0)),
                  ctr_spec, ctr_spec, ctr_spec],
        out_specs=pl.BlockSpec((1, rb * ns, c), lambda bi, si: (bi, si, 0)),
        out_shape=jax.ShapeDtypeStruct((b, s * ns, c), jnp.float32),
        compiler_params=pltpu.CompilerParams(
            dimension_semantics=("parallel", "parallel")),
    )(xs3, ys3, zs3, pts, cx3, cy3, cz3)


def _group2_kernel(xs_ref, ys_ref, zs_ref, pts_ref, cx_ref, cy_ref, cz_ref,
                   oa_ref, ob_ref, *, r2a, nsa, r2b, nsb, rb, nc, lw, c):
    xs = xs_ref[0].reshape(1, nc, lw)
    ys = ys_ref[0].reshape(1, nc, lw)
    zs = zs_ref[0].reshape(1, nc, lw)
    cxb = cx_ref[0].reshape(rb, 1, 1)
    cyb = cy_ref[0].reshape(rb, 1, 1)
    czb = cz_ref[0].reshape(rb, 1, 1)
    dx = cxb - xs
    dy = cyb - ys
    dz = czb - zs
    sqr = dx * dx + dy * dy + dz * dz

    lio = jax.lax.broadcasted_iota(jnp.int32, (lw, lw), 0)
    ljo = jax.lax.broadcasted_iota(jnp.int32, (lw, lw), 1)
    tri = (lio <= ljo).astype(jnp.float32)
    cio = jax.lax.broadcasted_iota(jnp.int32, (nc, nc), 0)
    cjo = jax.lax.broadcasted_iota(jnp.int32, (nc, nc), 1)
    tri_c = (cio <= cjo).astype(jnp.float32)
    lio2 = jax.lax.broadcasted_iota(jnp.int32, (1, lw), 1)

    for r2, ns, out_ref in ((r2a, nsa, oa_ref), (r2b, nsb, ob_ref)):
        mask2d = (sqr <= r2).reshape(rb * nc, lw)
        mask_f = mask2d.astype(jnp.float32)
        lr = jnp.dot(mask_f, tri, preferred_element_type=jnp.float32)
        lrm = jnp.where(mask2d, lr, 0.0)
        cc = lr[:, lw - 1:lw].reshape(rb, 1, nc)
        ci = jnp.dot(cc.reshape(rb, nc), tri_c,
                     preferred_element_type=jnp.float32).reshape(rb, 1, nc)
        ce = ci - cc
        count = ci[:, :, nc - 1:nc]
        kio = jax.lax.broadcasted_iota(
            jnp.int32, (rb, ns, 1), 1).astype(jnp.float32)
        t = jnp.where(kio < count, kio + 1.0, 1.0)
        oh1 = jnp.logical_and(t > ce, t <= ci).astype(jnp.float32)
        base = jnp.sum(oh1 * ce, axis=2, keepdims=True)
        lt = (t - base).reshape(rb * ns, 1)
        oh1_2d = oh1.reshape(rb * ns, nc)
        rio2 = jax.lax.broadcasted_iota(jnp.int32, (rb * ns, rb * nc), 0)
        qio2 = jax.lax.broadcasted_iota(jnp.int32, (rb * ns, rb * nc), 1)
        bd = jnp.where(rio2 // ns == qio2 // nc,
                       jnp.tile(oh1_2d, (1, rb)), 0.0)
        lrg = jnp.dot(bd, lrm, preferred_element_type=jnp.float32)
        lane = jnp.min(jnp.where(lrg == lt, lio2, lw), axis=1,
                       keepdims=True)
        pg = jnp.dot(oh1_2d, pts_ref[0],
                     preferred_element_type=jnp.float32,
                     precision=jax.lax.Precision.HIGHEST)
        grouped = jnp.concatenate(
            [jnp.take_along_axis(pg[:, ch * lw:(ch + 1) * lw], lane, axis=1)
             for ch in range(c)], axis=1)
        g3 = grouped.reshape(rb, ns, c)
        chio = jax.lax.broadcasted_iota(jnp.int32, (rb, ns, c), 2)
        ctr = (jnp.where(chio == 0, cxb, 0.0)
               + jnp.where(chio == 1, cyb, 0.0)
               + jnp.where(chio == 2, czb, 0.0))
        out_ref[0] = (g3 - ctr).reshape(rb * ns, c)


def _group2(xs, ys, zs, pts, cx, cy, cz, ra, nsa, rbb, nsb, rb, lw):
    b, n = xs.shape
    c = pts.shape[-1]
    s = cx.shape[1]
    nc = n // lw
    xs3 = xs.reshape(b, nc, lw)
    ys3 = ys.reshape(b, nc, lw)
    zs3 = zs.reshape(b, nc, lw)
    pts3 = pts.reshape(b, nc, lw, c).transpose(0, 1, 3, 2).reshape(
        b, nc, c * lw)
    cx3 = cx.reshape(b, s, 1)
    cy3 = cy.reshape(b, s, 1)
    cz3 = cz.reshape(b, s, 1)
    kern = functools.partial(_group2_kernel, r2a=ra * ra, nsa=nsa,
                             r2b=rbb * rbb, nsb=nsb, rb=rb, nc=nc, lw=lw, c=c)
    row_spec = pl.BlockSpec((1, nc, lw), lambda bi, si: (bi, 0, 0))
    ctr_spec = pl.BlockSpec((1, rb, 1), lambda bi, si: (bi, si, 0))
    return pl.pallas_call(
        kern,
        grid=(b, s // rb),
        in_specs=[row_spec, row_spec, row_spec,
                  pl.BlockSpec((1, nc, lw * c), lambda bi, si: (bi, 0, 0)),
                  ctr_spec, ctr_spec, ctr_spec],
        out_specs=[
            pl.BlockSpec((1, rb * nsa, c), lambda bi, si: (bi, si, 0)),
            pl.BlockSpec((1, rb * nsb, c), lambda bi, si: (bi, si, 0)),
        ],
        out_shape=[
            jax.ShapeDtypeStruct((b, s * nsa, c), jnp.float32),
            jax.ShapeDtypeStruct((b, s * nsb, c), jnp.float32),
        ],
        compiler_params=pltpu.CompilerParams(
            dimension_semantics=("parallel", "parallel")),
    )(xs3, ys3, zs3, pts3, cx3, cy3, cz3)


def _group2i_kernel(xs_ref, ys_ref, zs_ref, cx_ref, cy_ref, cz_ref,
                    oa_ref, ob_ref, *, r2a, nsa, r2b, nsb, rb, nc, lw, n):
    xs = xs_ref[0].reshape(1, nc, lw)
    ys = ys_ref[0].reshape(1, nc, lw)
    zs = zs_ref[0].reshape(1, nc, lw)
    cxb = cx_ref[0].reshape(rb, 1, 1)
    cyb = cy_ref[0].reshape(rb, 1, 1)
    czb = cz_ref[0].reshape(rb, 1, 1)
    dx = cxb - xs
    dy = cyb - ys
    dz = czb - zs
    sqr = dx * dx + dy * dy + dz * dz

    lio = jax.lax.broadcasted_iota(jnp.int32, (lw, lw), 0)
    ljo = jax.lax.broadcasted_iota(jnp.int32, (lw, lw), 1)
    tri = (lio <= ljo).astype(jnp.float32)
    cio = jax.lax.broadcasted_iota(jnp.int32, (nc, nc), 0)
    cjo = jax.lax.broadcasted_iota(jnp.int32, (nc, nc), 1)
    tri_c = (cio <= cjo).astype(jnp.float32)
    cfio = jax.lax.broadcasted_iota(
        jnp.int32, (rb, 1, nc), 2).astype(jnp.float32)
    lio2 = jax.lax.broadcasted_iota(jnp.int32, (1, lw), 1)
    goff = pl.program_id(0) * n

    for r2, ns, out_ref in ((r2a, nsa, oa_ref), (r2b, nsb, ob_ref)):
        mask2d = (sqr <= r2).reshape(rb * nc, lw)
        mask_f = mask2d.astype(jnp.float32)
        lr = jnp.dot(mask_f, tri, preferred_element_type=jnp.float32)
        lrm = jnp.where(mask2d, lr, 0.0)
        cc = lr[:, lw - 1:lw].reshape(rb, 1, nc)
        ci = jnp.dot(cc.reshape(rb, nc), tri_c,
                     preferred_element_type=jnp.float32).reshape(rb, 1, nc)
        ce = ci - cc
        count = ci[:, :, nc - 1:nc]
        kio = jax.lax.broadcasted_iota(
            jnp.int32, (rb, ns, 1), 1).astype(jnp.float32)
        t = jnp.where(kio < count, kio + 1.0, 1.0)
        oh1 = jnp.logical_and(t > ce, t <= ci).astype(jnp.float32)
        base = jnp.sum(oh1 * ce, axis=2, keepdims=True)
        cidx = jnp.sum(oh1 * cfio, axis=2, keepdims=True)
        lt = (t - base).reshape(rb * ns, 1)
        oh1_2d = oh1.reshape(rb * ns, nc)
        rio2 = jax.lax.broadcasted_iota(jnp.int32, (rb * ns, rb * nc), 0)
        qio2 = jax.lax.broadcasted_iota(jnp.int32, (rb * ns, rb * nc), 1)
        bd = jnp.where(rio2 // ns == qio2 // nc,
                       jnp.tile(oh1_2d, (1, rb)), 0.0)
        lrg = jnp.dot(bd, lrm, preferred_element_type=jnp.float32)
        lane = jnp.min(jnp.where(lrg == lt, lio2, lw), axis=1,
                       keepdims=True)
        j = cidx.reshape(rb * ns, 1).astype(jnp.int32) * lw + lane + goff
        out_ref[0] = j


def _group2i(xs, ys, zs, cx, cy, cz, ra, nsa, rbb, nsb, rb, lw):
    b, n = xs.shape
    s = cx.shape[1]
    nc = n // lw
    xs3 = xs.reshape(b, nc, lw)
    ys3 = ys.reshape(b, nc, lw)
    zs3 = zs.reshape(b, nc, lw)
    cx3 = cx.reshape(b, s, 1)
    cy3 = cy.reshape(b, s, 1)
    cz3 = cz.reshape(b, s, 1)
    kern = functools.partial(_group2i_kernel, r2a=ra * ra, nsa=nsa,
                             r2b=rbb * rbb, nsb=nsb, rb=rb, nc=nc, lw=lw, n=n)
    row_spec = pl.BlockSpec((1, nc, lw), lambda bi, si: (bi, 0, 0))
    ctr_spec = pl.BlockSpec((1, rb, 1), lambda bi, si: (bi, si, 0))
    return pl.pallas_call(
        kern,
        grid=(b, s // rb),
        in_specs=[row_spec, row_spec, row_spec,
                  ctr_spec, ctr_spec, ctr_spec],
        out_specs=[
            pl.BlockSpec((1, rb * nsa, 1), lambda bi, si: (bi, si, 0)),
            pl.BlockSpec((1, rb * nsb, 1), lambda bi, si: (bi, si, 0)),
        ],
        out_shape=[
            jax.ShapeDtypeStruct((b, s * nsa, 1), jnp.int32),
            jax.ShapeDtypeStruct((b, s * nsb, 1), jnp.int32),
        ],
        compiler_params=pltpu.CompilerParams(
            dimension_semantics=("parallel", "parallel")),
    )(xs3, ys3, zs3, cx3, cy3, cz3)


def _sc_gather(table, idx3, rows):
    d = table.shape[1]
    info = plsc.get_sparse_core_info()
    nw = info.num_cores * info.num_subcores
    b_per_w = rows // nw
    nchunk = b_per_w // 128
    mesh = plsc.VectorSubcoreMesh(core_axis_name="c", subcore_axis_name="s")

    nch_pad = max(nchunk, 8)

    @functools.partial(
        pl.kernel, mesh=mesh,
        out_type=jax.ShapeDtypeStruct((rows, d), jnp.float32),
        scratch_types=[
            pltpu.VMEM((nch_pad, 128), jnp.int32),
            pltpu.VMEM((128, d), jnp.float32),
            pltpu.SemaphoreType.DMA,
        ],
    )
    def k(table_hbm, idx_hbm, out_hbm, idx_v, rows_v, sem):
        wid = lax.axis_index("s") * info.num_cores + lax.axis_index("c")
        base = wid * b_per_w
        pltpu.sync_copy(idx_hbm.at[wid], idx_v)
        for j in range(nchunk):
            pltpu.async_copy(
                table_hbm.at[idx_v.at[j]], rows_v, sem).wait()
            pltpu.sync_copy(rows_v, out_hbm.at[pl.ds(base + j * 128, 128)])

    return k(table, idx3)


def _sc_idx_prep(idx, nw=32):
    flat = idx.reshape(-1)
    nchunk = flat.shape[0] // nw // 128
    i3 = flat.reshape(nw, nchunk, 128)
    if nchunk < 8:
        i3 = jnp.pad(i3, ((0, 0), (0, 8 - nchunk), (0, 0)))
    return i3


def _sc_gather_rows(table, idx, rows, max_chunks=16):
    per_call = 32 * 128 * max_chunks
    if rows <= per_call:
        return _sc_gather(table, _sc_idx_prep(idx), rows)
    flat = idx.reshape(-1)
    parts = [_sc_gather(table, _sc_idx_prep(flat[o:o + per_call]), per_call)
             for o in range(0, rows, per_call)]
    return jnp.concatenate(parts, axis=0)


def _mm_kernel(x_ref, w_ref, b_ref, y_ref, s1_ref, s2_ref):
    y = jnp.dot(x_ref[...], w_ref[...],
                preferred_element_type=jnp.float32) + b_ref[...]
    y_ref[...] = y
    p1 = jnp.sum(y, axis=0, keepdims=True)
    p2 = jnp.sum(y * y, axis=0, keepdims=True)

    @pl.when(pl.program_id(0) == 0)
    def _init():
        s1_ref[...] = p1
        s2_ref[...] = p2

    @pl.when(pl.program_id(0) > 0)
    def _acc():
        s1_ref[...] += p1
        s2_ref[...] += p2


def _mmbn_kernel(x_ref, s1i_ref, s2i_ref, g_ref, be_ref, w_ref, b_ref,
                 y_ref, s1_ref, s2_ref, *, inv_cnt):
    m = s1i_ref[...] * inv_cnt
    v = s2i_ref[...] * inv_cnt - m * m
    a = jnp.maximum(
        g_ref[...] * (x_ref[...] - m) / jnp.sqrt(v + 1e-5) + be_ref[...], 0.0)
    y = jnp.dot(a, w_ref[...], preferred_element_type=jnp.float32) + b_ref[...]
    y_ref[...] = y
    p1 = jnp.sum(y, axis=0, keepdims=True)
    p2 = jnp.sum(y * y, axis=0, keepdims=True)

    @pl.when(pl.program_id(0) == 0)
    def _init():
        s1_ref[...] = p1
        s2_ref[...] = p2

    @pl.when(pl.program_id(0) > 0)
    def _acc():
        s1_ref[...] += p1
        s2_ref[...] += p2


def _mmsub_kernel(x_ref, ctr_ref, w_ref, b_ref, y_ref, s1_ref, s2_ref):
    x = x_ref[...]
    ctr_pad = jnp.pad(ctr_ref[...], ((0, 0), (0, x.shape[1] - 3)))
    y = jnp.dot(x - ctr_pad, w_ref[...],
                preferred_element_type=jnp.float32) + b_ref[...]
    y_ref[...] = y
    p1 = jnp.sum(y, axis=0, keepdims=True)
    p2 = jnp.sum(y * y, axis=0, keepdims=True)

    @pl.when(pl.program_id(0) == 0)
    def _init():
        s1_ref[...] = p1
        s2_ref[...] = p2

    @pl.when(pl.program_id(0) > 0)
    def _acc():
        s1_ref[...] += p1
        s2_ref[...] += p2


def _mmsub(x, ctr, w, b, rblk):
    r, cin = x.shape
    cout = w.shape[1]
    wp = jnp.pad(w, ((0, cin - w.shape[0]), (0, 0)))
    out_specs, out_shape = _stats_out(r, cout, rblk)
    return pl.pallas_call(
        _mmsub_kernel,
        grid=(r // rblk,),
        in_specs=[pl.BlockSpec((rblk, cin), lambda i: (i, 0)),
                  pl.BlockSpec((rblk, 3), lambda i: (i, 0)),
                  pl.BlockSpec((cin, cout), lambda i: (0, 0)),
                  pl.BlockSpec((1, cout), lambda i: (0, 0))],
        out_specs=out_specs,
        out_shape=out_shape,
    )(x, ctr, wp, b.reshape(1, cout))


def _stats_out(r, cout, rblk):
    specs = [pl.BlockSpec((rblk, cout), lambda i: (i, 0)),
             pl.BlockSpec((1, cout), lambda i: (0, 0)),
             pl.BlockSpec((1, cout), lambda i: (0, 0))]
    shapes = [jax.ShapeDtypeStruct((r, cout), jnp.float32),
              jax.ShapeDtypeStruct((1, cout), jnp.float32),
              jax.ShapeDtypeStruct((1, cout), jnp.float32)]
    return specs, shapes


def _mm(x, w, b, rblk):
    r, cin = x.shape
    cout = w.shape[1]
    out_specs, out_shape = _stats_out(r, cout, rblk)
    return pl.pallas_call(
        _mm_kernel,
        grid=(r // rblk,),
        in_specs=[pl.BlockSpec((rblk, cin), lambda i: (i, 0)),
                  pl.BlockSpec((cin, cout), lambda i: (0, 0)),
                  pl.BlockSpec((1, cout), lambda i: (0, 0))],
        out_specs=out_specs,
        out_shape=out_shape,
    )(x, w, b.reshape(1, cout))


def _mmbn(x, s1, s2, g, be, w, b, inv_cnt, rblk):
    r, cin = x.shape
    cout = w.shape[1]
    out_specs, out_shape = _stats_out(r, cout, rblk)
    stat_spec = pl.BlockSpec((1, cin), lambda i: (0, 0))
    return pl.pallas_call(
        functools.partial(_mmbn_kernel, inv_cnt=inv_cnt),
        grid=(r // rblk,),
        in_specs=[pl.BlockSpec((rblk, cin), lambda i: (i, 0)),
                  stat_spec, stat_spec, stat_spec, stat_spec,
                  pl.BlockSpec((cin, cout), lambda i: (0, 0)),
                  pl.BlockSpec((1, cout), lambda i: (0, 0))],
        out_specs=out_specs,
        out_shape=out_shape,
    )(x, s1, s2, g.reshape(1, cin), be.reshape(1, cin), w, b.reshape(1, cout))


def _pool_kernel(y_ref, s1_ref, s2_ref, g_ref, be_ref, o_ref, *, inv_cnt):
    d = y_ref.shape[2]
    m = (s1_ref[...] * inv_cnt).reshape(1, 1, d)
    v = (s2_ref[...] * inv_cnt).reshape(1, 1, d) - m * m
    g = g_ref[...].reshape(1, 1, d)
    be = be_ref[...].reshape(1, 1, d)
    a = jnp.maximum(g * (y_ref[...] - m) / jnp.sqrt(v + 1e-5) + be, 0.0)
    o_ref[...] = jnp.max(a, axis=1)


def _pool(y3, s1, s2, g, be, inv_cnt, gb):
    rows, ns, d = y3.shape
    stat_spec = pl.BlockSpec((1, d), lambda i: (0, 0))
    return pl.pallas_call(
        functools.partial(_pool_kernel, inv_cnt=inv_cnt),
        grid=(rows // gb,),
        in_specs=[pl.BlockSpec((gb, ns, d), lambda i: (i, 0, 0)),
                  stat_spec, stat_spec, stat_spec, stat_spec],
        out_specs=pl.BlockSpec((gb, d), lambda i: (i, 0)),
        out_shape=jax.ShapeDtypeStruct((rows, d), jnp.float32),
        compiler_params=pltpu.CompilerParams(
            dimension_semantics=("parallel",)),
    )(y3, s1, s2, g.reshape(1, d), be.reshape(1, d))


def _sa_msg(xs, ys, zs, pts, npoint, specs, params, rb,
            sc_gather=False, lw=128):
    b, n = xs.shape
    cx, cy, cz = _fps_centers(xs, ys, zs, npoint)
    (ra, nsa, _), (rbb, nsb, _) = specs
    if sc_gather:
        ia, ib = _group2i(xs, ys, zs, cx, cy, cz, ra, nsa, rbb, nsb, rb, lw)
        c = pts.shape[-1]
        table = jnp.pad(pts.reshape(b * n, c), ((0, 0), (0, 128 - c)))
        groups = [_sc_gather_rows(table, ia, b * npoint * nsa),
                  _sc_gather_rows(table, ib, b * npoint * nsb)]
        ctr3 = jnp.stack([cx, cy, cz], axis=-1).reshape(b * npoint, 1, 3)
    else:
        groups = list(_group2(xs, ys, zs, pts, cx, cy, cz, ra, nsa, rbb, nsb,
                              rb, lw))
    outs = []
    for ((radius, ns, dims), mlp, grouped) in zip(specs, params, groups):
        inv_cnt = 1.0 / (b * npoint * ns)
        w, bb, g, be = mlp[0]
        if sc_gather:
            x = grouped
            ctr_rows = jnp.broadcast_to(ctr3, (b * npoint, ns, 3)).reshape(
                b * npoint * ns, 3)
            y, s1, s2 = _mmsub(x, ctr_rows, w, bb, rblk=512)
        else:
            x = grouped.reshape(b * npoint * ns, dims[0])
            y, s1, s2 = _mm(x, w, bb, rblk=512)
        for w2, b2, g2, be2 in mlp[1:]:
            y, s1n, s2n = _mmbn(y, s1, s2, g, be, w2, b2, inv_cnt, rblk=512)
            s1, s2, g, be = s1n, s2n, g2, be2
        pooled = _pool(y.reshape(b * npoint, ns, dims[-1]), s1, s2, g, be,
                       inv_cnt, gb=128)
        outs.append(pooled.reshape(b, npoint, dims[-1]))
    return (cx, cy, cz), jnp.concatenate(outs, axis=-1)


def kernel(pointcloud, params):
    b, n, _ = pointcloud.shape
    xs = pointcloud[..., 0]
    ys = pointcloud[..., 1]
    zs = pointcloud[..., 2]
    (cx1, cy1, cz1), f1 = _sa_msg(xs, ys, zs, pointcloud, _NPOINT1,
                                  _SA1_SPECS, params["sa1"], rb=32,
                                  sc_gather=True)
    pts2 = jnp.concatenate([jnp.stack([cx1, cy1, cz1], axis=-1), f1], axis=-1)
    (cx2, cy2, cz2), f2 = _sa_msg(cx1, cy1, cz1, pts2, _NPOINT2,
                                  _SA2_SPECS, params["sa2"], rb=32,
                                  sc_gather=True)
    lin, _, _ = _mm(f2.reshape(b * _NPOINT2, f2.shape[-1]),
                    params["linear_w"], params["linear_b"], rblk=512)
    xyz2 = jnp.stack([cx2, cy2, cz2], axis=-1)
    return jnp.concatenate([xyz2, lin.reshape(b, _NPOINT2, -1)], axis=-1)


# speedup vs baseline: 1.0652x; 1.0652x over previous
import functools

import jax
import jax.numpy as jnp
from jax import lax
from jax.experimental import pallas as pl
from jax.experimental.pallas import tpu as pltpu
from jax.experimental.pallas import tpu_sc as plsc

_SA1_SPECS = [(0.05, 16, [9, 16, 16, 32]), (0.1, 32, [9, 32, 32, 64])]
_SA2_SPECS = [(0.1, 16, [99, 64, 64, 128]), (0.2, 32, [99, 64, 96, 128])]
_NPOINT1, _NPOINT2 = 1024, 256


def _fps_kernel(xs_ref, ys_ref, zs_ref, cx_ref, cy_ref, cz_ref, dist_ref,
                *, npoint, n):
    b = xs_ref.shape[0]
    xs = xs_ref[...]
    ys = ys_ref[...]
    zs = zs_ref[...]
    col = jax.lax.broadcasted_iota(jnp.int32, (b, n), 1)
    colp = jax.lax.broadcasted_iota(jnp.int32, (b, npoint), 1)
    dist_ref[...] = jnp.full((b, n), 1e10, jnp.float32)
    cx_ref[...] = jnp.zeros((b, npoint), jnp.float32)
    cy_ref[...] = jnp.zeros((b, npoint), jnp.float32)
    cz_ref[...] = jnp.zeros((b, npoint), jnp.float32)

    def body(t, far):
        sel = col == far
        cx = jnp.sum(jnp.where(sel, xs, 0.0), axis=1, keepdims=True)
        cy = jnp.sum(jnp.where(sel, ys, 0.0), axis=1, keepdims=True)
        cz = jnp.sum(jnp.where(sel, zs, 0.0), axis=1, keepdims=True)
        hit = colp == t
        cx_ref[...] = jnp.where(hit, cx, cx_ref[...])
        cy_ref[...] = jnp.where(hit, cy, cy_ref[...])
        cz_ref[...] = jnp.where(hit, cz, cz_ref[...])
        dx = xs - cx
        dy = ys - cy
        dz = zs - cz
        d = dx * dx + dy * dy + dz * dz
        dist = jnp.minimum(dist_ref[...], d)
        dist_ref[...] = dist
        mx = jnp.max(dist, axis=1, keepdims=True)
        far_new = jnp.min(jnp.where(dist == mx, col, n), axis=1, keepdims=True)
        return far_new

    jax.lax.fori_loop(0, npoint, body, jnp.zeros((b, 1), jnp.int32))


def _fps_centers(xs, ys, zs, npoint):
    b, n = xs.shape
    out_shape = [jax.ShapeDtypeStruct((b, npoint), jnp.float32)] * 3
    return pl.pallas_call(
        functools.partial(_fps_kernel, npoint=npoint, n=n),
        out_shape=out_shape,
        scratch_shapes=[pltpu.VMEM((b, n), jnp.float32)],
    )(xs, ys, zs)


def _group2i_kernel(xs_ref, ys_ref, zs_ref, cx_ref, cy_ref, cz_ref,
                    oa_ref, ob_ref, *, r2a, nsa, r2b, nsb, rb, nc, lw, n):
    xs = xs_ref[0].reshape(1, nc, lw)
    ys = ys_ref[0].reshape(1, nc, lw)
    zs = zs_ref[0].reshape(1, nc, lw)
    cxb = cx_ref[0].reshape(rb, 1, 1)
    cyb = cy_ref[0].reshape(rb, 1, 1)
    czb = cz_ref[0].reshape(rb, 1, 1)
    dx = cxb - xs
    dy = cyb - ys
    dz = czb - zs
    sqr = dx * dx + dy * dy + dz * dz

    lio = jax.lax.broadcasted_iota(jnp.int32, (lw, lw), 0)
    ljo = jax.lax.broadcasted_iota(jnp.int32, (lw, lw), 1)
    tri = (lio <= ljo).astype(jnp.float32)
    cio = jax.lax.broadcasted_iota(jnp.int32, (nc, nc), 0)
    cjo = jax.lax.broadcasted_iota(jnp.int32, (nc, nc), 1)
    tri_c = (cio <= cjo).astype(jnp.float32)
    cfio = jax.lax.broadcasted_iota(
        jnp.int32, (rb, 1, nc), 2).astype(jnp.float32)
    lio2 = jax.lax.broadcasted_iota(jnp.int32, (1, lw), 1)
    goff = pl.program_id(0) * n

    for r2, ns, out_ref in ((r2a, nsa, oa_ref), (r2b, nsb, ob_ref)):
        mask2d = (sqr <= r2).reshape(rb * nc, lw)
        mask_f = mask2d.astype(jnp.float32)
        lr = jnp.dot(mask_f, tri, preferred_element_type=jnp.float32)
        lrm = jnp.where(mask2d, lr, 0.0)
        cc = lr[:, lw - 1:lw].reshape(rb, 1, nc)
        ci = jnp.dot(cc.reshape(rb, nc), tri_c,
                     preferred_element_type=jnp.float32).reshape(rb, 1, nc)
        ce = ci - cc
        count = ci[:, :, nc - 1:nc]
        kio = jax.lax.broadcasted_iota(
            jnp.int32, (rb, ns, 1), 1).astype(jnp.float32)
        t = jnp.where(kio < count, kio + 1.0, 1.0)
        oh1 = jnp.logical_and(t > ce, t <= ci).astype(jnp.float32)
        base = jnp.sum(oh1 * ce, axis=2, keepdims=True)
        cidx = jnp.sum(oh1 * cfio, axis=2, keepdims=True)
        lt = (t - base).reshape(rb * ns, 1)
        oh1_2d = oh1.reshape(rb * ns, nc)
        rio2 = jax.lax.broadcasted_iota(jnp.int32, (rb * ns, rb * nc), 0)
        qio2 = jax.lax.broadcasted_iota(jnp.int32, (rb * ns, rb * nc), 1)
        bd = jnp.where(rio2 // ns == qio2 // nc,
                       jnp.tile(oh1_2d, (1, rb)), 0.0)
        lrg = jnp.dot(bd, lrm, preferred_element_type=jnp.float32)
        lane = jnp.min(jnp.where(lrg == lt, lio2, lw), axis=1,
                       keepdims=True)
        j = cidx.reshape(rb * ns, 1).astype(jnp.int32) * lw + lane + goff
        out_ref[0] = j


def _group2i(xs, ys, zs, cx, cy, cz, ra, nsa, rbb, nsb, rb, lw):
    b, n = xs.shape
    s = cx.shape[1]
    nc = n // lw
    xs3 = xs.reshape(b, nc, lw)
    ys3 = ys.reshape(b, nc, lw)
    zs3 = zs.reshape(b, nc, lw)
    cx3 = cx.reshape(b, s, 1)
    cy3 = cy.reshape(b, s, 1)
    cz3 = cz.reshape(b, s, 1)
    kern = functools.partial(_group2i_kernel, r2a=ra * ra, nsa=nsa,
                             r2b=rbb * rbb, nsb=nsb, rb=rb, nc=nc, lw=lw, n=n)
    row_spec = pl.BlockSpec((1, nc, lw), lambda bi, si: (bi, 0, 0))
    ctr_spec = pl.BlockSpec((1, rb, 1), lambda bi, si: (bi, si, 0))
    return pl.pallas_call(
        kern,
        grid=(b, s // rb),
        in_specs=[row_spec, row_spec, row_spec,
                  ctr_spec, ctr_spec, ctr_spec],
        out_specs=[
            pl.BlockSpec((1, rb * nsa, 1), lambda bi, si: (bi, si, 0)),
            pl.BlockSpec((1, rb * nsb, 1), lambda bi, si: (bi, si, 0)),
        ],
        out_shape=[
            jax.ShapeDtypeStruct((b, s * nsa, 1), jnp.int32),
            jax.ShapeDtypeStruct((b, s * nsb, 1), jnp.int32),
        ],
        compiler_params=pltpu.CompilerParams(
            dimension_semantics=("parallel", "parallel")),
    )(xs3, ys3, zs3, cx3, cy3, cz3)


def _sc_gather(table, idx3, rows):
    d = table.shape[1]
    info = plsc.get_sparse_core_info()
    nw = info.num_cores * info.num_subcores
    b_per_w = rows // nw
    nchunk = b_per_w // 128
    mesh = plsc.VectorSubcoreMesh(core_axis_name="c", subcore_axis_name="s")

    nch_pad = max(nchunk, 8)

    @functools.partial(
        pl.kernel, mesh=mesh,
        out_type=jax.ShapeDtypeStruct((rows, d), jnp.float32),
        scratch_types=[
            pltpu.VMEM((nch_pad, 128), jnp.int32),
            pltpu.VMEM((128, d), jnp.float32),
            pltpu.SemaphoreType.DMA,
        ],
    )
    def k(table_hbm, idx_hbm, out_hbm, idx_v, rows_v, sem):
        wid = lax.axis_index("s") * info.num_cores + lax.axis_index("c")
        base = wid * b_per_w
        pltpu.sync_copy(idx_hbm.at[wid], idx_v)
        for j in range(nchunk):
            pltpu.async_copy(
                table_hbm.at[idx_v.at[j]], rows_v, sem).wait()
            pltpu.sync_copy(rows_v, out_hbm.at[pl.ds(base + j * 128, 128)])

    return k(table, idx3)


def _sc_idx_prep(idx, nw=32):
    flat = idx.reshape(-1)
    nchunk = flat.shape[0] // nw // 128
    i3 = flat.reshape(nw, nchunk, 128)
    if nchunk < 8:
        i3 = jnp.pad(i3, ((0, 0), (0, 8 - nchunk), (0, 0)))
    return i3


def _sc_gather_rows(table, idx, rows, max_chunks=16):
    per_call = 32 * 128 * max_chunks
    if rows <= per_call:
        return _sc_gather(table, _sc_idx_prep(idx), rows)
    flat = idx.reshape(-1)
    parts = [_sc_gather(table, _sc_idx_prep(flat[o:o + per_call]), per_call)
             for o in range(0, rows, per_call)]
    return jnp.concatenate(parts, axis=0)


def _mm_kernel(x_ref, w_ref, b_ref, y_ref, s1_ref, s2_ref):
    y = jnp.dot(x_ref[...], w_ref[...],
                preferred_element_type=jnp.float32) + b_ref[...]
    y_ref[...] = y
    p1 = jnp.sum(y, axis=0, keepdims=True)
    p2 = jnp.sum(y * y, axis=0, keepdims=True)

    @pl.when(pl.program_id(0) == 0)
    def _init():
        s1_ref[...] = p1
        s2_ref[...] = p2

    @pl.when(pl.program_id(0) > 0)
    def _acc():
        s1_ref[...] += p1
        s2_ref[...] += p2


def _mmbn_kernel(x_ref, s1i_ref, s2i_ref, g_ref, be_ref, w_ref, b_ref,
                 y_ref, s1_ref, s2_ref, *, inv_cnt):
    m = s1i_ref[...] * inv_cnt
    v = s2i_ref[...] * inv_cnt - m * m
    a = jnp.maximum(
        g_ref[...] * (x_ref[...] - m) / jnp.sqrt(v + 1e-5) + be_ref[...], 0.0)
    y = jnp.dot(a, w_ref[...], preferred_element_type=jnp.float32) + b_ref[...]
    y_ref[...] = y
    p1 = jnp.sum(y, axis=0, keepdims=True)
    p2 = jnp.sum(y * y, axis=0, keepdims=True)

    @pl.when(pl.program_id(0) == 0)
    def _init():
        s1_ref[...] = p1
        s2_ref[...] = p2

    @pl.when(pl.program_id(0) > 0)
    def _acc():
        s1_ref[...] += p1
        s2_ref[...] += p2


def _mmsub_kernel(x_ref, ctr_ref, w_ref, b_ref, y_ref, s1_ref, s2_ref):
    x = x_ref[...]
    ctr_pad = jnp.pad(ctr_ref[...], ((0, 0), (0, x.shape[1] - 3)))
    y = jnp.dot(x - ctr_pad, w_ref[...],
                preferred_element_type=jnp.float32) + b_ref[...]
    y_ref[...] = y
    p1 = jnp.sum(y, axis=0, keepdims=True)
    p2 = jnp.sum(y * y, axis=0, keepdims=True)

    @pl.when(pl.program_id(0) == 0)
    def _init():
        s1_ref[...] = p1
        s2_ref[...] = p2

    @pl.when(pl.program_id(0) > 0)
    def _acc():
        s1_ref[...] += p1
        s2_ref[...] += p2


def _mmsub(x, ctr, w, b, rblk):
    r, cin = x.shape
    cout = w.shape[1]
    wp = jnp.pad(w, ((0, cin - w.shape[0]), (0, 0)))
    out_specs, out_shape = _stats_out(r, cout, rblk)
    return pl.pallas_call(
        _mmsub_kernel,
        grid=(r // rblk,),
        in_specs=[pl.BlockSpec((rblk, cin), lambda i: (i, 0)),
                  pl.BlockSpec((rblk, 3), lambda i: (i, 0)),
                  pl.BlockSpec((cin, cout), lambda i: (0, 0)),
                  pl.BlockSpec((1, cout), lambda i: (0, 0))],
        out_specs=out_specs,
        out_shape=out_shape,
    )(x, ctr, wp, b.reshape(1, cout))


def _stats_out(r, cout, rblk):
    specs = [pl.BlockSpec((rblk, cout), lambda i: (i, 0)),
             pl.BlockSpec((1, cout), lambda i: (0, 0)),
             pl.BlockSpec((1, cout), lambda i: (0, 0))]
    shapes = [jax.ShapeDtypeStruct((r, cout), jnp.float32),
              jax.ShapeDtypeStruct((1, cout), jnp.float32),
              jax.ShapeDtypeStruct((1, cout), jnp.float32)]
    return specs, shapes


def _mm(x, w, b, rblk):
    r, cin = x.shape
    cout = w.shape[1]
    out_specs, out_shape = _stats_out(r, cout, rblk)
    return pl.pallas_call(
        _mm_kernel,
        grid=(r // rblk,),
        in_specs=[pl.BlockSpec((rblk, cin), lambda i: (i, 0)),
                  pl.BlockSpec((cin, cout), lambda i: (0, 0)),
                  pl.BlockSpec((1, cout), lambda i: (0, 0))],
        out_specs=out_specs,
        out_shape=out_shape,
    )(x, w, b.reshape(1, cout))


def _mmbn(x, s1, s2, g, be, w, b, inv_cnt, rblk):
    r, cin = x.shape
    cout = w.shape[1]
    out_specs, out_shape = _stats_out(r, cout, rblk)
    stat_spec = pl.BlockSpec((1, cin), lambda i: (0, 0))
    return pl.pallas_call(
        functools.partial(_mmbn_kernel, inv_cnt=inv_cnt),
        grid=(r // rblk,),
        in_specs=[pl.BlockSpec((rblk, cin), lambda i: (i, 0)),
                  stat_spec, stat_spec, stat_spec, stat_spec,
                  pl.BlockSpec((cin, cout), lambda i: (0, 0)),
                  pl.BlockSpec((1, cout), lambda i: (0, 0))],
        out_specs=out_specs,
        out_shape=out_shape,
    )(x, s1, s2, g.reshape(1, cin), be.reshape(1, cin), w, b.reshape(1, cout))


def _pool_kernel(y_ref, s1_ref, s2_ref, g_ref, be_ref, o_ref, *, inv_cnt):
    d = y_ref.shape[2]
    m = (s1_ref[...] * inv_cnt).reshape(1, 1, d)
    v = (s2_ref[...] * inv_cnt).reshape(1, 1, d) - m * m
    g = g_ref[...].reshape(1, 1, d)
    be = be_ref[...].reshape(1, 1, d)
    a = jnp.maximum(g * (y_ref[...] - m) / jnp.sqrt(v + 1e-5) + be, 0.0)
    o_ref[...] = jnp.max(a, axis=1)


def _pool(y3, s1, s2, g, be, inv_cnt, gb):
    rows, ns, d = y3.shape
    stat_spec = pl.BlockSpec((1, d), lambda i: (0, 0))
    return pl.pallas_call(
        functools.partial(_pool_kernel, inv_cnt=inv_cnt),
        grid=(rows // gb,),
        in_specs=[pl.BlockSpec((gb, ns, d), lambda i: (i, 0, 0)),
                  stat_spec, stat_spec, stat_spec, stat_spec],
        out_specs=pl.BlockSpec((gb, d), lambda i: (i, 0)),
        out_shape=jax.ShapeDtypeStruct((rows, d), jnp.float32),
        compiler_params=pltpu.CompilerParams(
            dimension_semantics=("parallel",)),
    )(y3, s1, s2, g.reshape(1, d), be.reshape(1, d))


def _sa_msg(xs, ys, zs, pts, npoint, specs, params, rb, lw=128):
    b, n = xs.shape
    cx, cy, cz = _fps_centers(xs, ys, zs, npoint)
    (ra, nsa, _), (rbb, nsb, _) = specs
    ia, ib = _group2i(xs, ys, zs, cx, cy, cz, ra, nsa, rbb, nsb, rb, lw)
    c = pts.shape[-1]
    table = jnp.pad(pts.reshape(b * n, c), ((0, 0), (0, 128 - c)))
    groups = [_sc_gather_rows(table, ia, b * npoint * nsa),
              _sc_gather_rows(table, ib, b * npoint * nsb)]
    ctr3 = jnp.stack([cx, cy, cz], axis=-1).reshape(b * npoint, 1, 3)
    outs = []
    for ((radius, ns, dims), mlp, grouped) in zip(specs, params, groups):
        inv_cnt = 1.0 / (b * npoint * ns)
        w, bb, g, be = mlp[0]
        ctr_rows = jnp.broadcast_to(ctr3, (b * npoint, ns, 3)).reshape(
            b * npoint * ns, 3)
        y, s1, s2 = _mmsub(grouped, ctr_rows, w, bb, rblk=512)
        for w2, b2, g2, be2 in mlp[1:]:
            y, s1n, s2n = _mmbn(y, s1, s2, g, be, w2, b2, inv_cnt, rblk=512)
            s1, s2, g, be = s1n, s2n, g2, be2
        pooled = _pool(y.reshape(b * npoint, ns, dims[-1]), s1, s2, g, be,
                       inv_cnt, gb=128)
        outs.append(pooled.reshape(b, npoint, dims[-1]))
    return (cx, cy, cz), jnp.concatenate(outs, axis=-1)


def kernel(pointcloud, params):
    b, n, _ = pointcloud.shape
    xs = pointcloud[..., 0]
    ys = pointcloud[..., 1]
    zs = pointcloud[..., 2]
    (cx1, cy1, cz1), f1 = _sa_msg(xs, ys, zs, pointcloud, _NPOINT1,
                                  _SA1_SPECS, params["sa1"], rb=16)
    pts2 = jnp.concatenate([jnp.stack([cx1, cy1, cz1], axis=-1), f1], axis=-1)
    (cx2, cy2, cz2), f2 = _sa_msg(cx1, cy1, cz1, pts2, _NPOINT2,
                                  _SA2_SPECS, params["sa2"], rb=16)
    lin, _, _ = _mm(f2.reshape(b * _NPOINT2, f2.shape[-1]),
                    params["linear_w"], params["linear_b"], rblk=512)
    xyz2 = jnp.stack([cx2, cy2, cz2], axis=-1)
    return jnp.concatenate([xyz2, lin.reshape(b, _NPOINT2, -1)], axis=-1)


# speedup vs baseline: 1.3343x; 1.2526x over previous
import functools

import jax
import jax.numpy as jnp
from jax import lax
from jax.experimental import pallas as pl
from jax.experimental.pallas import tpu as pltpu
from jax.experimental.pallas import tpu_sc as plsc

_SA1_SPECS = [(0.05, 16, [9, 16, 16, 32]), (0.1, 32, [9, 32, 32, 64])]
_SA2_SPECS = [(0.1, 16, [99, 64, 64, 128]), (0.2, 32, [99, 64, 96, 128])]
_NPOINT1, _NPOINT2 = 1024, 256


def _fps_kernel(xs_ref, ys_ref, zs_ref, cx_ref, cy_ref, cz_ref, dist_ref,
                *, npoint, n):
    b = xs_ref.shape[0]
    xs = xs_ref[...]
    ys = ys_ref[...]
    zs = zs_ref[...]
    col = jax.lax.broadcasted_iota(jnp.int32, (b, n), 1)
    colp = jax.lax.broadcasted_iota(jnp.int32, (b, npoint), 1)
    dist_ref[...] = jnp.full((b, n), 1e10, jnp.float32)
    cx_ref[...] = jnp.zeros((b, npoint), jnp.float32)
    cy_ref[...] = jnp.zeros((b, npoint), jnp.float32)
    cz_ref[...] = jnp.zeros((b, npoint), jnp.float32)

    def body(t, far):
        sel = col == far
        cx = jnp.sum(jnp.where(sel, xs, 0.0), axis=1, keepdims=True)
        cy = jnp.sum(jnp.where(sel, ys, 0.0), axis=1, keepdims=True)
        cz = jnp.sum(jnp.where(sel, zs, 0.0), axis=1, keepdims=True)
        hit = colp == t
        cx_ref[...] = jnp.where(hit, cx, cx_ref[...])
        cy_ref[...] = jnp.where(hit, cy, cy_ref[...])
        cz_ref[...] = jnp.where(hit, cz, cz_ref[...])
        dx = xs - cx
        dy = ys - cy
        dz = zs - cz
        d = dx * dx + dy * dy + dz * dz
        dist = jnp.minimum(dist_ref[...], d)
        dist_ref[...] = dist
        mx = jnp.max(dist, axis=1, keepdims=True)
        far_new = jnp.min(jnp.where(dist == mx, col, n), axis=1, keepdims=True)
        return far_new

    jax.lax.fori_loop(0, npoint, body, jnp.zeros((b, 1), jnp.int32))


def _fps_centers(xs, ys, zs, npoint):
    b, n = xs.shape
    out_shape = [jax.ShapeDtypeStruct((b, npoint), jnp.float32)] * 3
    return pl.pallas_call(
        functools.partial(_fps_kernel, npoint=npoint, n=n),
        out_shape=out_shape,
        scratch_shapes=[pltpu.VMEM((b, n), jnp.float32)],
    )(xs, ys, zs)


def _group2i_kernel(xs_ref, ys_ref, zs_ref, cx_ref, cy_ref, cz_ref,
                    oa_ref, ob_ref, *, r2a, nsa, r2b, nsb, rb, nc, lw, n):
    xs = xs_ref[0].reshape(1, nc, lw)
    ys = ys_ref[0].reshape(1, nc, lw)
    zs = zs_ref[0].reshape(1, nc, lw)
    cxb = cx_ref[0].reshape(rb, 1, 1)
    cyb = cy_ref[0].reshape(rb, 1, 1)
    czb = cz_ref[0].reshape(rb, 1, 1)
    dx = cxb - xs
    dy = cyb - ys
    dz = czb - zs
    sqr = dx * dx + dy * dy + dz * dz

    lio = jax.lax.broadcasted_iota(jnp.int32, (lw, lw), 0)
    ljo = jax.lax.broadcasted_iota(jnp.int32, (lw, lw), 1)
    tri = (lio <= ljo).astype(jnp.float32)
    cio = jax.lax.broadcasted_iota(jnp.int32, (nc, nc), 0)
    cjo = jax.lax.broadcasted_iota(jnp.int32, (nc, nc), 1)
    tri_c = (cio <= cjo).astype(jnp.float32)
    cfio = jax.lax.broadcasted_iota(
        jnp.int32, (rb, 1, nc), 2).astype(jnp.float32)
    lio2 = jax.lax.broadcasted_iota(jnp.int32, (1, lw), 1)
    goff = pl.program_id(0) * n

    for r2, ns, out_ref in ((r2a, nsa, oa_ref), (r2b, nsb, ob_ref)):
        mask2d = (sqr <= r2).reshape(rb * nc, lw)
        mask_f = mask2d.astype(jnp.float32)
        lr = jnp.dot(mask_f, tri, preferred_element_type=jnp.float32)
        lrm = jnp.where(mask2d, lr, 0.0)
        cc = lr[:, lw - 1:lw].reshape(rb, 1, nc)
        ci = jnp.dot(cc.reshape(rb, nc), tri_c,
                     preferred_element_type=jnp.float32).reshape(rb, 1, nc)
        ce = ci - cc
        count = ci[:, :, nc - 1:nc]
        kio = jax.lax.broadcasted_iota(
            jnp.int32, (rb, ns, 1), 1).astype(jnp.float32)
        t = jnp.where(kio < count, kio + 1.0, 1.0)
        oh1 = jnp.logical_and(t > ce, t <= ci).astype(jnp.float32)
        base = jnp.sum(oh1 * ce, axis=2, keepdims=True)
        cidx = jnp.sum(oh1 * cfio, axis=2, keepdims=True)
        lt = (t - base).reshape(rb * ns, 1)
        oh1_2d = oh1.reshape(rb * ns, nc)
        rio2 = jax.lax.broadcasted_iota(jnp.int32, (rb * ns, rb * nc), 0)
        qio2 = jax.lax.broadcasted_iota(jnp.int32, (rb * ns, rb * nc), 1)
        bd = jnp.where(rio2 // ns == qio2 // nc,
                       jnp.tile(oh1_2d, (1, rb)), 0.0)
        lrg = jnp.dot(bd, lrm, preferred_element_type=jnp.float32)
        lane = jnp.min(jnp.where(lrg == lt, lio2, lw), axis=1,
                       keepdims=True)
        j = cidx.reshape(rb * ns, 1).astype(jnp.int32) * lw + lane + goff
        out_ref[0] = j


def _group2i(xs, ys, zs, cx, cy, cz, ra, nsa, rbb, nsb, rb, lw):
    b, n = xs.shape
    s = cx.shape[1]
    nc = n // lw
    xs3 = xs.reshape(b, nc, lw)
    ys3 = ys.reshape(b, nc, lw)
    zs3 = zs.reshape(b, nc, lw)
    cx3 = cx.reshape(b, s, 1)
    cy3 = cy.reshape(b, s, 1)
    cz3 = cz.reshape(b, s, 1)
    kern = functools.partial(_group2i_kernel, r2a=ra * ra, nsa=nsa,
                             r2b=rbb * rbb, nsb=nsb, rb=rb, nc=nc, lw=lw, n=n)
    row_spec = pl.BlockSpec((1, nc, lw), lambda bi, si: (bi, 0, 0))
    ctr_spec = pl.BlockSpec((1, rb, 1), lambda bi, si: (bi, si, 0))
    return pl.pallas_call(
        kern,
        grid=(b, s // rb),
        in_specs=[row_spec, row_spec, row_spec,
                  ctr_spec, ctr_spec, ctr_spec],
        out_specs=[
            pl.BlockSpec((1, rb * nsa, 1), lambda bi, si: (bi, si, 0)),
            pl.BlockSpec((1, rb * nsb, 1), lambda bi, si: (bi, si, 0)),
        ],
        out_shape=[
            jax.ShapeDtypeStruct((b, s * nsa, 1), jnp.int32),
            jax.ShapeDtypeStruct((b, s * nsb, 1), jnp.int32),
        ],
        compiler_params=pltpu.CompilerParams(
            dimension_semantics=("parallel", "parallel")),
    )(xs3, ys3, zs3, cx3, cy3, cz3)


def _sc_gather(table, idx3, rows):
    d = table.shape[1]
    info = plsc.get_sparse_core_info()
    nw = info.num_cores * info.num_subcores
    b_per_w = rows // nw
    nchunk = b_per_w // 128
    mesh = plsc.VectorSubcoreMesh(core_axis_name="c", subcore_axis_name="s")

    nch_pad = max(nchunk, 8)

    @functools.partial(
        pl.kernel, mesh=mesh,
        out_type=jax.ShapeDtypeStruct((rows, d), jnp.float32),
        scratch_types=[
            pltpu.VMEM((nch_pad, 128), jnp.int32),
            pltpu.VMEM((128, d), jnp.float32),
            pltpu.SemaphoreType.DMA,
        ],
    )
    def k(table_hbm, idx_hbm, out_hbm, idx_v, rows_v, sem):
        wid = lax.axis_index("s") * info.num_cores + lax.axis_index("c")
        base = wid * b_per_w
        pltpu.sync_copy(idx_hbm.at[wid], idx_v)
        for j in range(nchunk):
            pltpu.async_copy(
                table_hbm.at[idx_v.at[j]], rows_v, sem).wait()
            pltpu.sync_copy(rows_v, out_hbm.at[pl.ds(base + j * 128, 128)])

    return k(table, idx3)


def _sc_idx_prep(idx, nw=32):
    flat = idx.reshape(-1)
    nchunk = flat.shape[0] // nw // 128
    i3 = flat.reshape(nw, nchunk, 128)
    if nchunk < 8:
        i3 = jnp.pad(i3, ((0, 0), (0, 8 - nchunk), (0, 0)))
    return i3


def _sc_gather_rows(table, idx, rows, max_chunks=16):
    per_call = 32 * 128 * max_chunks
    if rows <= per_call:
        return _sc_gather(table, _sc_idx_prep(idx), rows)
    flat = idx.reshape(-1)
    parts = [_sc_gather(table, _sc_idx_prep(flat[o:o + per_call]), per_call)
             for o in range(0, rows, per_call)]
    return jnp.concatenate(parts, axis=0)


def _mm_kernel(x_ref, w_ref, b_ref, y_ref, s1_ref, s2_ref):
    y = jnp.dot(x_ref[...], w_ref[...],
                preferred_element_type=jnp.float32) + b_ref[...]
    y_ref[...] = y
    p1 = jnp.sum(y, axis=0, keepdims=True)
    p2 = jnp.sum(y * y, axis=0, keepdims=True)

    @pl.when(pl.program_id(0) == 0)
    def _init():
        s1_ref[...] = p1
        s2_ref[...] = p2

    @pl.when(pl.program_id(0) > 0)
    def _acc():
        s1_ref[...] += p1
        s2_ref[...] += p2


def _mmbn_kernel(x_ref, s1i_ref, s2i_ref, g_ref, be_ref, w_ref, b_ref,
                 y_ref, s1_ref, s2_ref, *, inv_cnt):
    m = s1i_ref[...] * inv_cnt
    v = s2i_ref[...] * inv_cnt - m * m
    a = jnp.maximum(
        g_ref[...] * (x_ref[...] - m) / jnp.sqrt(v + 1e-5) + be_ref[...], 0.0)
    y = jnp.dot(a, w_ref[...], preferred_element_type=jnp.float32) + b_ref[...]
    y_ref[...] = y
    p1 = jnp.sum(y, axis=0, keepdims=True)
    p2 = jnp.sum(y * y, axis=0, keepdims=True)

    @pl.when(pl.program_id(0) == 0)
    def _init():
        s1_ref[...] = p1
        s2_ref[...] = p2

    @pl.when(pl.program_id(0) > 0)
    def _acc():
        s1_ref[...] += p1
        s2_ref[...] += p2


def _mmsub_kernel(x_ref, ctr_ref, w_ref, b_ref, y_ref, s1_ref, s2_ref):
    x = x_ref[...]
    ctr_pad = jnp.pad(ctr_ref[...], ((0, 0), (0, x.shape[1] - 3)))
    y = jnp.dot(x - ctr_pad, w_ref[...],
                preferred_element_type=jnp.float32) + b_ref[...]
    y_ref[...] = y
    p1 = jnp.sum(y, axis=0, keepdims=True)
    p2 = jnp.sum(y * y, axis=0, keepdims=True)

    @pl.when(pl.program_id(0) == 0)
    def _init():
        s1_ref[...] = p1
        s2_ref[...] = p2

    @pl.when(pl.program_id(0) > 0)
    def _acc():
        s1_ref[...] += p1
        s2_ref[...] += p2


def _mmsub(x, ctr, w, b, rblk):
    r, cin = x.shape
    cout = w.shape[1]
    wp = jnp.pad(w, ((0, cin - w.shape[0]), (0, 0)))
    out_specs, out_shape = _stats_out(r, cout, rblk)
    return pl.pallas_call(
        _mmsub_kernel,
        grid=(r // rblk,),
        in_specs=[pl.BlockSpec((rblk, cin), lambda i: (i, 0)),
                  pl.BlockSpec((rblk, 3), lambda i: (i, 0)),
                  pl.BlockSpec((cin, cout), lambda i: (0, 0)),
                  pl.BlockSpec((1, cout), lambda i: (0, 0))],
        out_specs=out_specs,
        out_shape=out_shape,
    )(x, ctr, wp, b.reshape(1, cout))


def _stats_out(r, cout, rblk):
    specs = [pl.BlockSpec((rblk, cout), lambda i: (i, 0)),
             pl.BlockSpec((1, cout), lambda i: (0, 0)),
             pl.BlockSpec((1, cout), lambda i: (0, 0))]
    shapes = [jax.ShapeDtypeStruct((r, cout), jnp.float32),
              jax.ShapeDtypeStruct((1, cout), jnp.float32),
              jax.ShapeDtypeStruct((1, cout), jnp.float32)]
    return specs, shapes


def _mm(x, w, b, rblk):
    r, cin = x.shape
    cout = w.shape[1]
    out_specs, out_shape = _stats_out(r, cout, rblk)
    return pl.pallas_call(
        _mm_kernel,
        grid=(r // rblk,),
        in_specs=[pl.BlockSpec((rblk, cin), lambda i: (i, 0)),
                  pl.BlockSpec((cin, cout), lambda i: (0, 0)),
                  pl.BlockSpec((1, cout), lambda i: (0, 0))],
        out_specs=out_specs,
        out_shape=out_shape,
    )(x, w, b.reshape(1, cout))


def _mmbn(x, s1, s2, g, be, w, b, inv_cnt, rblk):
    r, cin = x.shape
    cout = w.shape[1]
    out_specs, out_shape = _stats_out(r, cout, rblk)
    stat_spec = pl.BlockSpec((1, cin), lambda i: (0, 0))
    return pl.pallas_call(
        functools.partial(_mmbn_kernel, inv_cnt=inv_cnt),
        grid=(r // rblk,),
        in_specs=[pl.BlockSpec((rblk, cin), lambda i: (i, 0)),
                  stat_spec, stat_spec, stat_spec, stat_spec,
                  pl.BlockSpec((cin, cout), lambda i: (0, 0)),
                  pl.BlockSpec((1, cout), lambda i: (0, 0))],
        out_specs=out_specs,
        out_shape=out_shape,
    )(x, s1, s2, g.reshape(1, cin), be.reshape(1, cin), w, b.reshape(1, cout))


def _pool_kernel(y_ref, s1_ref, s2_ref, g_ref, be_ref, o_ref, *, inv_cnt):
    d = y_ref.shape[2]
    m = (s1_ref[...] * inv_cnt).reshape(1, 1, d)
    v = (s2_ref[...] * inv_cnt).reshape(1, 1, d) - m * m
    g = g_ref[...].reshape(1, 1, d)
    be = be_ref[...].reshape(1, 1, d)
    a = jnp.maximum(g * (y_ref[...] - m) / jnp.sqrt(v + 1e-5) + be, 0.0)
    o_ref[...] = jnp.max(a, axis=1)


def _pool(y3, s1, s2, g, be, inv_cnt, gb):
    rows, ns, d = y3.shape
    stat_spec = pl.BlockSpec((1, d), lambda i: (0, 0))
    return pl.pallas_call(
        functools.partial(_pool_kernel, inv_cnt=inv_cnt),
        grid=(rows // gb,),
        in_specs=[pl.BlockSpec((gb, ns, d), lambda i: (i, 0, 0)),
                  stat_spec, stat_spec, stat_spec, stat_spec],
        out_specs=pl.BlockSpec((gb, d), lambda i: (i, 0)),
        out_shape=jax.ShapeDtypeStruct((rows, d), jnp.float32),
        compiler_params=pltpu.CompilerParams(
            dimension_semantics=("parallel",)),
    )(y3, s1, s2, g.reshape(1, d), be.reshape(1, d))


def _sa_msg(xs, ys, zs, pts, npoint, specs, params, rb, lw=128):
    b, n = xs.shape
    cx, cy, cz = _fps_centers(xs, ys, zs, npoint)
    (ra, nsa, _), (rbb, nsb, _) = specs
    ia, ib = _group2i(xs, ys, zs, cx, cy, cz, ra, nsa, rbb, nsb, rb, lw)
    c = pts.shape[-1]
    table = jnp.pad(pts.reshape(b * n, c), ((0, 0), (0, 128 - c)))
    groups = [_sc_gather_rows(table, ia, b * npoint * nsa),
              _sc_gather_rows(table, ib, b * npoint * nsb)]
    ctr3 = jnp.stack([cx, cy, cz], axis=-1).reshape(b * npoint, 1, 3)
    outs = []
    for ((radius, ns, dims), mlp, grouped) in zip(specs, params, groups):
        inv_cnt = 1.0 / (b * npoint * ns)
        w, bb, g, be = mlp[0]
        ctr_rows = jnp.broadcast_to(ctr3, (b * npoint, ns, 3)).reshape(
            b * npoint * ns, 3)
        y, s1, s2 = _mmsub(grouped, ctr_rows, w, bb, rblk=2048)
        for w2, b2, g2, be2 in mlp[1:]:
            y, s1n, s2n = _mmbn(y, s1, s2, g, be, w2, b2, inv_cnt, rblk=2048)
            s1, s2, g, be = s1n, s2n, g2, be2
        pooled = _pool(y.reshape(b * npoint, ns, dims[-1]), s1, s2, g, be,
                       inv_cnt, gb=256)
        outs.append(pooled.reshape(b, npoint, dims[-1]))
    return (cx, cy, cz), jnp.concatenate(outs, axis=-1)


def kernel(pointcloud, params):
    b, n, _ = pointcloud.shape
    xs = pointcloud[..., 0]
    ys = pointcloud[..., 1]
    zs = pointcloud[..., 2]
    (cx1, cy1, cz1), f1 = _sa_msg(xs, ys, zs, pointcloud, _NPOINT1,
                                  _SA1_SPECS, params["sa1"], rb=16)
    pts2 = jnp.concatenate([jnp.stack([cx1, cy1, cz1], axis=-1), f1], axis=-1)
    (cx2, cy2, cz2), f2 = _sa_msg(cx1, cy1, cz1, pts2, _NPOINT2,
                                  _SA2_SPECS, params["sa2"], rb=16)
    lin, _, _ = _mm(f2.reshape(b * _NPOINT2, f2.shape[-1]),
                    params["linear_w"], params["linear_b"], rblk=512)
    xyz2 = jnp.stack([cx2, cy2, cz2], axis=-1)
    return jnp.concatenate([xyz2, lin.reshape(b, _NPOINT2, -1)], axis=-1)


# speedup vs baseline: 1.4016x; 1.0505x over previous
import functools

import jax
import jax.numpy as jnp
from jax import lax
from jax.experimental import pallas as pl
from jax.experimental.pallas import tpu as pltpu
from jax.experimental.pallas import tpu_sc as plsc

_SA1_SPECS = [(0.05, 16, [9, 16, 16, 32]), (0.1, 32, [9, 32, 32, 64])]
_SA2_SPECS = [(0.1, 16, [99, 64, 64, 128]), (0.2, 32, [99, 64, 96, 128])]
_NPOINT1, _NPOINT2 = 1024, 256


def _fps_kernel(xs_ref, ys_ref, zs_ref, cx_ref, cy_ref, cz_ref, dist_ref,
                *, npoint, n):
    b = xs_ref.shape[0]
    xs = xs_ref[...]
    ys = ys_ref[...]
    zs = zs_ref[...]
    col = jax.lax.broadcasted_iota(jnp.int32, (b, n), 1)
    colp = jax.lax.broadcasted_iota(jnp.int32, (b, npoint), 1)
    dist_ref[...] = jnp.full((b, n), 1e10, jnp.float32)
    cx_ref[...] = jnp.zeros((b, npoint), jnp.float32)
    cy_ref[...] = jnp.zeros((b, npoint), jnp.float32)
    cz_ref[...] = jnp.zeros((b, npoint), jnp.float32)

    def body(t, far):
        sel = col == far
        cx = jnp.sum(jnp.where(sel, xs, 0.0), axis=1, keepdims=True)
        cy = jnp.sum(jnp.where(sel, ys, 0.0), axis=1, keepdims=True)
        cz = jnp.sum(jnp.where(sel, zs, 0.0), axis=1, keepdims=True)
        hit = colp == t
        cx_ref[...] = jnp.where(hit, cx, cx_ref[...])
        cy_ref[...] = jnp.where(hit, cy, cy_ref[...])
        cz_ref[...] = jnp.where(hit, cz, cz_ref[...])
        dx = xs - cx
        dy = ys - cy
        dz = zs - cz
        d = dx * dx + dy * dy + dz * dz
        dist = jnp.minimum(dist_ref[...], d)
        dist_ref[...] = dist
        mx = jnp.max(dist, axis=1, keepdims=True)
        far_new = jnp.min(jnp.where(dist == mx, col, n), axis=1, keepdims=True)
        return far_new

    jax.lax.fori_loop(0, npoint, body, jnp.zeros((b, 1), jnp.int32))


def _fps_centers(xs, ys, zs, npoint):
    b, n = xs.shape
    out_shape = [jax.ShapeDtypeStruct((b, npoint), jnp.float32)] * 3
    return pl.pallas_call(
        functools.partial(_fps_kernel, npoint=npoint, n=n),
        out_shape=out_shape,
        scratch_shapes=[pltpu.VMEM((b, n), jnp.float32)],
    )(xs, ys, zs)


def _group2i_kernel(xs_ref, ys_ref, zs_ref, cx_ref, cy_ref, cz_ref,
                    oa_ref, ob_ref, *, r2a, nsa, r2b, nsb, rb, nc, lw, n):
    xs = xs_ref[0].reshape(1, nc, lw)
    ys = ys_ref[0].reshape(1, nc, lw)
    zs = zs_ref[0].reshape(1, nc, lw)
    cxb = cx_ref[0].reshape(rb, 1, 1)
    cyb = cy_ref[0].reshape(rb, 1, 1)
    czb = cz_ref[0].reshape(rb, 1, 1)
    dx = cxb - xs
    dy = cyb - ys
    dz = czb - zs
    sqr = dx * dx + dy * dy + dz * dz

    lio = jax.lax.broadcasted_iota(jnp.int32, (lw, lw), 0)
    ljo = jax.lax.broadcasted_iota(jnp.int32, (lw, lw), 1)
    tri = (lio <= ljo).astype(jnp.float32)
    cio = jax.lax.broadcasted_iota(jnp.int32, (nc, nc), 0)
    cjo = jax.lax.broadcasted_iota(jnp.int32, (nc, nc), 1)
    tri_c = (cio <= cjo).astype(jnp.float32)
    cfio = jax.lax.broadcasted_iota(
        jnp.int32, (rb, 1, nc), 2).astype(jnp.float32)
    lio2 = jax.lax.broadcasted_iota(jnp.int32, (1, lw), 1)
    goff = pl.program_id(0) * n

    for r2, ns, out_ref in ((r2a, nsa, oa_ref), (r2b, nsb, ob_ref)):
        mask2d = (sqr <= r2).reshape(rb * nc, lw)
        mask_f = mask2d.astype(jnp.float32)
        lr = jnp.dot(mask_f, tri, preferred_element_type=jnp.float32)
        lrm = jnp.where(mask2d, lr, 0.0)
        cc = lr[:, lw - 1:lw].reshape(rb, 1, nc)
        ci = jnp.dot(cc.reshape(rb, nc), tri_c,
                     preferred_element_type=jnp.float32).reshape(rb, 1, nc)
        ce = ci - cc
        count = ci[:, :, nc - 1:nc]
        kio = jax.lax.broadcasted_iota(
            jnp.int32, (rb, ns, 1), 1).astype(jnp.float32)
        t = jnp.where(kio < count, kio + 1.0, 1.0)
        oh1 = jnp.logical_and(t > ce, t <= ci).astype(jnp.float32)
        base = jnp.sum(oh1 * ce, axis=2, keepdims=True)
        cidx = jnp.sum(oh1 * cfio, axis=2, keepdims=True)
        lt = (t - base).reshape(rb * ns, 1)
        oh1_2d = oh1.reshape(rb * ns, nc)
        rio2 = jax.lax.broadcasted_iota(jnp.int32, (rb * ns, rb * nc), 0)
        qio2 = jax.lax.broadcasted_iota(jnp.int32, (rb * ns, rb * nc), 1)
        bd = jnp.where(rio2 // ns == qio2 // nc,
                       jnp.tile(oh1_2d, (1, rb)), 0.0)
        lrg = jnp.dot(bd, lrm, preferred_element_type=jnp.float32)
        lane = jnp.min(jnp.where(lrg == lt, lio2, lw), axis=1,
                       keepdims=True)
        j = cidx.reshape(rb * ns, 1).astype(jnp.int32) * lw + lane + goff
        out_ref[0] = j


def _group2i(xs, ys, zs, cx, cy, cz, ra, nsa, rbb, nsb, rb, lw):
    b, n = xs.shape
    s = cx.shape[1]
    nc = n // lw
    xs3 = xs.reshape(b, nc, lw)
    ys3 = ys.reshape(b, nc, lw)
    zs3 = zs.reshape(b, nc, lw)
    cx3 = cx.reshape(b, s, 1)
    cy3 = cy.reshape(b, s, 1)
    cz3 = cz.reshape(b, s, 1)
    kern = functools.partial(_group2i_kernel, r2a=ra * ra, nsa=nsa,
                             r2b=rbb * rbb, nsb=nsb, rb=rb, nc=nc, lw=lw, n=n)
    row_spec = pl.BlockSpec((1, nc, lw), lambda bi, si: (bi, 0, 0))
    ctr_spec = pl.BlockSpec((1, rb, 1), lambda bi, si: (bi, si, 0))
    return pl.pallas_call(
        kern,
        grid=(b, s // rb),
        in_specs=[row_spec, row_spec, row_spec,
                  ctr_spec, ctr_spec, ctr_spec],
        out_specs=[
            pl.BlockSpec((1, rb * nsa, 1), lambda bi, si: (bi, si, 0)),
            pl.BlockSpec((1, rb * nsb, 1), lambda bi, si: (bi, si, 0)),
        ],
        out_shape=[
            jax.ShapeDtypeStruct((b, s * nsa, 1), jnp.int32),
            jax.ShapeDtypeStruct((b, s * nsb, 1), jnp.int32),
        ],
        compiler_params=pltpu.CompilerParams(
            dimension_semantics=("parallel", "parallel")),
    )(xs3, ys3, zs3, cx3, cy3, cz3)


def _sc_gather(table, idx3, rows):
    d = table.shape[1]
    info = plsc.get_sparse_core_info()
    nw = info.num_cores * info.num_subcores
    b_per_w = rows // nw
    nchunk = b_per_w // 128
    mesh = plsc.VectorSubcoreMesh(core_axis_name="c", subcore_axis_name="s")

    nch_pad = max(nchunk, 8)

    @functools.partial(
        pl.kernel, mesh=mesh,
        out_type=jax.ShapeDtypeStruct((rows, d), jnp.float32),
        scratch_types=[
            pltpu.VMEM((nch_pad, 128), jnp.int32),
            pltpu.VMEM((128, d), jnp.float32),
            pltpu.SemaphoreType.DMA,
        ],
    )
    def k(table_hbm, idx_hbm, out_hbm, idx_v, rows_v, sem):
        wid = lax.axis_index("s") * info.num_cores + lax.axis_index("c")
        base = wid * b_per_w
        pltpu.sync_copy(idx_hbm.at[wid], idx_v)
        for j in range(nchunk):
            pltpu.async_copy(
                table_hbm.at[idx_v.at[j]], rows_v, sem).wait()
            pltpu.sync_copy(rows_v, out_hbm.at[pl.ds(base + j * 128, 128)])

    return k(table, idx3)


def _sc_idx_prep(idx, nw=32):
    flat = idx.reshape(-1)
    nchunk = flat.shape[0] // nw // 128
    i3 = flat.reshape(nw, nchunk, 128)
    if nchunk < 8:
        i3 = jnp.pad(i3, ((0, 0), (0, 8 - nchunk), (0, 0)))
    return i3


def _sc_gather_rows(table, idx, rows, max_chunks=16):
    per_call = 32 * 128 * max_chunks
    if rows <= per_call:
        return _sc_gather(table, _sc_idx_prep(idx), rows)
    flat = idx.reshape(-1)
    parts = [_sc_gather(table, _sc_idx_prep(flat[o:o + per_call]), per_call)
             for o in range(0, rows, per_call)]
    return jnp.concatenate(parts, axis=0)


def _mm_kernel(x_ref, w_ref, b_ref, y_ref, s1_ref, s2_ref):
    y = jnp.dot(x_ref[...], w_ref[...],
                preferred_element_type=jnp.float32) + b_ref[...]
    y_ref[...] = y
    p1 = jnp.sum(y, axis=0, keepdims=True)
    p2 = jnp.sum(y * y, axis=0, keepdims=True)

    @pl.when(pl.program_id(0) == 0)
    def _init():
        s1_ref[...] = p1
        s2_ref[...] = p2

    @pl.when(pl.program_id(0) > 0)
    def _acc():
        s1_ref[...] += p1
        s2_ref[...] += p2


def _mmbn_kernel(x_ref, s1i_ref, s2i_ref, g_ref, be_ref, w_ref, b_ref,
                 y_ref, s1_ref, s2_ref, *, inv_cnt):
    m = s1i_ref[...] * inv_cnt
    v = s2i_ref[...] * inv_cnt - m * m
    a = jnp.maximum(
        g_ref[...] * (x_ref[...] - m) / jnp.sqrt(v + 1e-5) + be_ref[...], 0.0)
    y = jnp.dot(a, w_ref[...], preferred_element_type=jnp.float32) + b_ref[...]
    y_ref[...] = y
    p1 = jnp.sum(y, axis=0, keepdims=True)
    p2 = jnp.sum(y * y, axis=0, keepdims=True)

    @pl.when(pl.program_id(0) == 0)
    def _init():
        s1_ref[...] = p1
        s2_ref[...] = p2

    @pl.when(pl.program_id(0) > 0)
    def _acc():
        s1_ref[...] += p1
        s2_ref[...] += p2


def _mmsub_kernel(x_ref, ctr_ref, w_ref, b_ref, y_ref, s1_ref, s2_ref):
    x = x_ref[...]
    ctr_pad = jnp.pad(ctr_ref[...], ((0, 0), (0, x.shape[1] - 3)))
    y = jnp.dot(x - ctr_pad, w_ref[...],
                preferred_element_type=jnp.float32) + b_ref[...]
    y_ref[...] = y
    p1 = jnp.sum(y, axis=0, keepdims=True)
    p2 = jnp.sum(y * y, axis=0, keepdims=True)

    @pl.when(pl.program_id(0) == 0)
    def _init():
        s1_ref[...] = p1
        s2_ref[...] = p2

    @pl.when(pl.program_id(0) > 0)
    def _acc():
        s1_ref[...] += p1
        s2_ref[...] += p2


def _mmsub(x, ctr, w, b, rblk):
    r, cin = x.shape
    cout = w.shape[1]
    wp = jnp.pad(w, ((0, cin - w.shape[0]), (0, 0)))
    out_specs, out_shape = _stats_out(r, cout, rblk)
    return pl.pallas_call(
        _mmsub_kernel,
        grid=(r // rblk,),
        in_specs=[pl.BlockSpec((rblk, cin), lambda i: (i, 0)),
                  pl.BlockSpec((rblk, 3), lambda i: (i, 0)),
                  pl.BlockSpec((cin, cout), lambda i: (0, 0)),
                  pl.BlockSpec((1, cout), lambda i: (0, 0))],
        out_specs=out_specs,
        out_shape=out_shape,
    )(x, ctr, wp, b.reshape(1, cout))


def _stats_out(r, cout, rblk):
    specs = [pl.BlockSpec((rblk, cout), lambda i: (i, 0)),
             pl.BlockSpec((1, cout), lambda i: (0, 0)),
             pl.BlockSpec((1, cout), lambda i: (0, 0))]
    shapes = [jax.ShapeDtypeStruct((r, cout), jnp.float32),
              jax.ShapeDtypeStruct((1, cout), jnp.float32),
              jax.ShapeDtypeStruct((1, cout), jnp.float32)]
    return specs, shapes


def _mm(x, w, b, rblk):
    r, cin = x.shape
    cout = w.shape[1]
    out_specs, out_shape = _stats_out(r, cout, rblk)
    return pl.pallas_call(
        _mm_kernel,
        grid=(r // rblk,),
        in_specs=[pl.BlockSpec((rblk, cin), lambda i: (i, 0)),
                  pl.BlockSpec((cin, cout), lambda i: (0, 0)),
                  pl.BlockSpec((1, cout), lambda i: (0, 0))],
        out_specs=out_specs,
        out_shape=out_shape,
    )(x, w, b.reshape(1, cout))


def _mmbn(x, s1, s2, g, be, w, b, inv_cnt, rblk):
    r, cin = x.shape
    cout = w.shape[1]
    out_specs, out_shape = _stats_out(r, cout, rblk)
    stat_spec = pl.BlockSpec((1, cin), lambda i: (0, 0))
    return pl.pallas_call(
        functools.partial(_mmbn_kernel, inv_cnt=inv_cnt),
        grid=(r // rblk,),
        in_specs=[pl.BlockSpec((rblk, cin), lambda i: (i, 0)),
                  stat_spec, stat_spec, stat_spec, stat_spec,
                  pl.BlockSpec((cin, cout), lambda i: (0, 0)),
                  pl.BlockSpec((1, cout), lambda i: (0, 0))],
        out_specs=out_specs,
        out_shape=out_shape,
    )(x, s1, s2, g.reshape(1, cin), be.reshape(1, cin), w, b.reshape(1, cout))


def _pool_kernel(y_ref, s1_ref, s2_ref, g_ref, be_ref, o_ref, *, inv_cnt):
    d = y_ref.shape[2]
    m = (s1_ref[...] * inv_cnt).reshape(1, 1, d)
    v = (s2_ref[...] * inv_cnt).reshape(1, 1, d) - m * m
    g = g_ref[...].reshape(1, 1, d)
    be = be_ref[...].reshape(1, 1, d)
    a = jnp.maximum(g * (y_ref[...] - m) / jnp.sqrt(v + 1e-5) + be, 0.0)
    o_ref[...] = jnp.max(a, axis=1)


def _pool(y3, s1, s2, g, be, inv_cnt, gb):
    rows, ns, d = y3.shape
    stat_spec = pl.BlockSpec((1, d), lambda i: (0, 0))
    return pl.pallas_call(
        functools.partial(_pool_kernel, inv_cnt=inv_cnt),
        grid=(rows // gb,),
        in_specs=[pl.BlockSpec((gb, ns, d), lambda i: (i, 0, 0)),
                  stat_spec, stat_spec, stat_spec, stat_spec],
        out_specs=pl.BlockSpec((gb, d), lambda i: (i, 0)),
        out_shape=jax.ShapeDtypeStruct((rows, d), jnp.float32),
        compiler_params=pltpu.CompilerParams(
            dimension_semantics=("parallel",)),
    )(y3, s1, s2, g.reshape(1, d), be.reshape(1, d))


def _sa_msg(xs, ys, zs, pts, npoint, specs, params, rb, lw=128):
    b, n = xs.shape
    cx, cy, cz = _fps_centers(xs, ys, zs, npoint)
    (ra, nsa, _), (rbb, nsb, _) = specs
    ia, ib = _group2i(xs, ys, zs, cx, cy, cz, ra, nsa, rbb, nsb, rb, lw)
    c = pts.shape[-1]
    table = jnp.pad(pts.reshape(b * n, c), ((0, 0), (0, 128 - c)))
    groups = [_sc_gather_rows(table, ia, b * npoint * nsa),
              _sc_gather_rows(table, ib, b * npoint * nsb)]
    ctr3 = jnp.stack([cx, cy, cz], axis=-1).reshape(b * npoint, 1, 3)
    outs = []
    for ((radius, ns, dims), mlp, grouped) in zip(specs, params, groups):
        inv_cnt = 1.0 / (b * npoint * ns)
        w, bb, g, be = mlp[0]
        ctr_rows = jnp.broadcast_to(ctr3, (b * npoint, ns, 3)).reshape(
            b * npoint * ns, 3)
        y, s1, s2 = _mmsub(grouped, ctr_rows, w, bb, rblk=4096)
        for w2, b2, g2, be2 in mlp[1:]:
            y, s1n, s2n = _mmbn(y, s1, s2, g, be, w2, b2, inv_cnt, rblk=4096)
            s1, s2, g, be = s1n, s2n, g2, be2
        pooled = _pool(y.reshape(b * npoint, ns, dims[-1]), s1, s2, g, be,
                       inv_cnt, gb=512)
        outs.append(pooled.reshape(b, npoint, dims[-1]))
    return (cx, cy, cz), jnp.concatenate(outs, axis=-1)


def kernel(pointcloud, params):
    b, n, _ = pointcloud.shape
    xs = pointcloud[..., 0]
    ys = pointcloud[..., 1]
    zs = pointcloud[..., 2]
    (cx1, cy1, cz1), f1 = _sa_msg(xs, ys, zs, pointcloud, _NPOINT1,
                                  _SA1_SPECS, params["sa1"], rb=16)
    pts2 = jnp.concatenate([jnp.stack([cx1, cy1, cz1], axis=-1), f1], axis=-1)
    (cx2, cy2, cz2), f2 = _sa_msg(cx1, cy1, cz1, pts2, _NPOINT2,
                                  _SA2_SPECS, params["sa2"], rb=16)
    lin, _, _ = _mm(f2.reshape(b * _NPOINT2, f2.shape[-1]),
                    params["linear_w"], params["linear_b"], rblk=512)
    xyz2 = jnp.stack([cx2, cy2, cz2], axis=-1)
    return jnp.concatenate([xyz2, lin.reshape(b, _NPOINT2, -1)], axis=-1)


# speedup vs baseline: 1.4243x; 1.0162x over previous
import functools

import jax
import jax.numpy as jnp
from jax import lax
from jax.experimental import pallas as pl
from jax.experimental.pallas import tpu as pltpu
from jax.experimental.pallas import tpu_sc as plsc

_SA1_SPECS = [(0.05, 16, [9, 16, 16, 32]), (0.1, 32, [9, 32, 32, 64])]
_SA2_SPECS = [(0.1, 16, [99, 64, 64, 128]), (0.2, 32, [99, 64, 96, 128])]
_NPOINT1, _NPOINT2 = 1024, 256


def _fps_kernel(xs_ref, ys_ref, zs_ref, cx_ref, cy_ref, cz_ref, dist_ref,
                *, npoint, n):
    b = xs_ref.shape[0]
    xs = xs_ref[...]
    ys = ys_ref[...]
    zs = zs_ref[...]
    col = jax.lax.broadcasted_iota(jnp.int32, (b, n), 1)
    colp = jax.lax.broadcasted_iota(jnp.int32, (b, npoint), 1)
    dist_ref[...] = jnp.full((b, n), 1e10, jnp.float32)
    cx_ref[...] = jnp.zeros((b, npoint), jnp.float32)
    cy_ref[...] = jnp.zeros((b, npoint), jnp.float32)
    cz_ref[...] = jnp.zeros((b, npoint), jnp.float32)

    def body(t, far):
        sel = col == far
        cx = jnp.sum(jnp.where(sel, xs, 0.0), axis=1, keepdims=True)
        cy = jnp.sum(jnp.where(sel, ys, 0.0), axis=1, keepdims=True)
        cz = jnp.sum(jnp.where(sel, zs, 0.0), axis=1, keepdims=True)
        hit = colp == t
        cx_ref[...] = jnp.where(hit, cx, cx_ref[...])
        cy_ref[...] = jnp.where(hit, cy, cy_ref[...])
        cz_ref[...] = jnp.where(hit, cz, cz_ref[...])
        dx = xs - cx
        dy = ys - cy
        dz = zs - cz
        d = dx * dx + dy * dy + dz * dz
        dist = jnp.minimum(dist_ref[...], d)
        dist_ref[...] = dist
        mx = jnp.max(dist, axis=1, keepdims=True)
        far_new = jnp.min(jnp.where(dist == mx, col, n), axis=1, keepdims=True)
        return far_new

    jax.lax.fori_loop(0, npoint, body, jnp.zeros((b, 1), jnp.int32))


def _fps_centers(xs, ys, zs, npoint):
    b, n = xs.shape
    out_shape = [jax.ShapeDtypeStruct((b, npoint), jnp.float32)] * 3
    return pl.pallas_call(
        functools.partial(_fps_kernel, npoint=npoint, n=n),
        out_shape=out_shape,
        scratch_shapes=[pltpu.VMEM((b, n), jnp.float32)],
    )(xs, ys, zs)


def _group2i_kernel(xs_ref, ys_ref, zs_ref, cx_ref, cy_ref, cz_ref,
                    oa_ref, ob_ref, *, r2a, nsa, r2b, nsb, rb, nc, lw, n):
    xs = xs_ref[0].reshape(1, nc, lw)
    ys = ys_ref[0].reshape(1, nc, lw)
    zs = zs_ref[0].reshape(1, nc, lw)
    cxb = cx_ref[0].reshape(rb, 1, 1)
    cyb = cy_ref[0].reshape(rb, 1, 1)
    czb = cz_ref[0].reshape(rb, 1, 1)
    dx = cxb - xs
    dy = cyb - ys
    dz = czb - zs
    sqr = dx * dx + dy * dy + dz * dz

    lio = jax.lax.broadcasted_iota(jnp.int32, (lw, lw), 0)
    ljo = jax.lax.broadcasted_iota(jnp.int32, (lw, lw), 1)
    tri = (lio <= ljo).astype(jnp.float32)
    cio = jax.lax.broadcasted_iota(jnp.int32, (nc, nc), 0)
    cjo = jax.lax.broadcasted_iota(jnp.int32, (nc, nc), 1)
    tri_c = (cio <= cjo).astype(jnp.float32)
    cfio = jax.lax.broadcasted_iota(
        jnp.int32, (rb, 1, nc), 2).astype(jnp.float32)
    lio2 = jax.lax.broadcasted_iota(jnp.int32, (1, lw), 1)
    goff = pl.program_id(0) * n

    for r2, ns, out_ref in ((r2a, nsa, oa_ref), (r2b, nsb, ob_ref)):
        mask2d = (sqr <= r2).reshape(rb * nc, lw)
        mask_f = mask2d.astype(jnp.float32)
        lr = jnp.dot(mask_f, tri, preferred_element_type=jnp.float32)
        lrm = jnp.where(mask2d, lr, 0.0)
        cc = lr[:, lw - 1:lw].reshape(rb, 1, nc)
        ci = jnp.dot(cc.reshape(rb, nc), tri_c,
                     preferred_element_type=jnp.float32).reshape(rb, 1, nc)
        ce = ci - cc
        count = ci[:, :, nc - 1:nc]
        kio = jax.lax.broadcasted_iota(
            jnp.int32, (rb, ns, 1), 1).astype(jnp.float32)
        t = jnp.where(kio < count, kio + 1.0, 1.0)
        oh1 = jnp.logical_and(t > ce, t <= ci).astype(jnp.float32)
        base = jnp.sum(oh1 * ce, axis=2, keepdims=True)
        cidx = jnp.sum(oh1 * cfio, axis=2, keepdims=True)
        lt = (t - base).reshape(rb * ns, 1)
        oh1_2d = oh1.reshape(rb * ns, nc)
        rio2 = jax.lax.broadcasted_iota(jnp.int32, (rb * ns, rb * nc), 0)
        qio2 = jax.lax.broadcasted_iota(jnp.int32, (rb * ns, rb * nc), 1)
        bd = jnp.where(rio2 // ns == qio2 // nc,
                       jnp.tile(oh1_2d, (1, rb)), 0.0)
        lrg = jnp.dot(bd, lrm, preferred_element_type=jnp.float32)
        lane = jnp.min(jnp.where(lrg == lt, lio2, lw), axis=1,
                       keepdims=True)
        j = cidx.reshape(rb * ns, 1).astype(jnp.int32) * lw + lane + goff
        out_ref[0] = j


def _group2i(xs, ys, zs, cx, cy, cz, ra, nsa, rbb, nsb, rb, lw):
    b, n = xs.shape
    s = cx.shape[1]
    nc = n // lw
    xs3 = xs.reshape(b, nc, lw)
    ys3 = ys.reshape(b, nc, lw)
    zs3 = zs.reshape(b, nc, lw)
    cx3 = cx.reshape(b, s, 1)
    cy3 = cy.reshape(b, s, 1)
    cz3 = cz.reshape(b, s, 1)
    kern = functools.partial(_group2i_kernel, r2a=ra * ra, nsa=nsa,
                             r2b=rbb * rbb, nsb=nsb, rb=rb, nc=nc, lw=lw, n=n)
    row_spec = pl.BlockSpec((1, nc, lw), lambda bi, si: (bi, 0, 0))
    ctr_spec = pl.BlockSpec((1, rb, 1), lambda bi, si: (bi, si, 0))
    return pl.pallas_call(
        kern,
        grid=(b, s // rb),
        in_specs=[row_spec, row_spec, row_spec,
                  ctr_spec, ctr_spec, ctr_spec],
        out_specs=[
            pl.BlockSpec((1, rb * nsa, 1), lambda bi, si: (bi, si, 0)),
            pl.BlockSpec((1, rb * nsb, 1), lambda bi, si: (bi, si, 0)),
        ],
        out_shape=[
            jax.ShapeDtypeStruct((b, s * nsa, 1), jnp.int32),
            jax.ShapeDtypeStruct((b, s * nsb, 1), jnp.int32),
        ],
        compiler_params=pltpu.CompilerParams(
            dimension_semantics=("parallel", "parallel")),
    )(xs3, ys3, zs3, cx3, cy3, cz3)


def _sc_gather(table, idx3, rows):
    d = table.shape[1]
    info = plsc.get_sparse_core_info()
    nw = info.num_cores * info.num_subcores
    b_per_w = rows // nw
    nchunk = b_per_w // 128
    mesh = plsc.VectorSubcoreMesh(core_axis_name="c", subcore_axis_name="s")

    nch_pad = max(nchunk, 8)

    @functools.partial(
        pl.kernel, mesh=mesh,
        out_type=jax.ShapeDtypeStruct((rows, d), jnp.float32),
        scratch_types=[
            pltpu.VMEM((nch_pad, 128), jnp.int32),
            pltpu.VMEM((128, d), jnp.float32),
            pltpu.SemaphoreType.DMA,
        ],
    )
    def k(table_hbm, idx_hbm, out_hbm, idx_v, rows_v, sem):
        wid = lax.axis_index("s") * info.num_cores + lax.axis_index("c")
        base = wid * b_per_w
        pltpu.sync_copy(idx_hbm.at[wid], idx_v)
        for j in range(nchunk):
            pltpu.async_copy(
                table_hbm.at[idx_v.at[j]], rows_v, sem).wait()
            pltpu.sync_copy(rows_v, out_hbm.at[pl.ds(base + j * 128, 128)])

    return k(table, idx3)


def _sc_idx_prep(idx, nw=32):
    flat = idx.reshape(-1)
    nchunk = flat.shape[0] // nw // 128
    i3 = flat.reshape(nw, nchunk, 128)
    if nchunk < 8:
        i3 = jnp.pad(i3, ((0, 0), (0, 8 - nchunk), (0, 0)))
    return i3


def _sc_gather_rows(table, idx, rows, max_chunks=16):
    per_call = 32 * 128 * max_chunks
    if rows <= per_call:
        return _sc_gather(table, _sc_idx_prep(idx), rows)
    flat = idx.reshape(-1)
    parts = [_sc_gather(table, _sc_idx_prep(flat[o:o + per_call]), per_call)
             for o in range(0, rows, per_call)]
    return jnp.concatenate(parts, axis=0)


def _mm_kernel(x_ref, w_ref, b_ref, y_ref, s1_ref, s2_ref):
    y = jnp.dot(x_ref[...], w_ref[...],
                preferred_element_type=jnp.float32) + b_ref[...]
    y_ref[...] = y
    p1 = jnp.sum(y, axis=0, keepdims=True)
    p2 = jnp.sum(y * y, axis=0, keepdims=True)

    @pl.when(pl.program_id(0) == 0)
    def _init():
        s1_ref[...] = p1
        s2_ref[...] = p2

    @pl.when(pl.program_id(0) > 0)
    def _acc():
        s1_ref[...] += p1
        s2_ref[...] += p2


def _mmbn_kernel(x_ref, s1i_ref, s2i_ref, g_ref, be_ref, w_ref, b_ref,
                 y_ref, s1_ref, s2_ref, *, inv_cnt):
    m = s1i_ref[...] * inv_cnt
    v = s2i_ref[...] * inv_cnt - m * m
    a = jnp.maximum(
        g_ref[...] * (x_ref[...] - m) / jnp.sqrt(v + 1e-5) + be_ref[...], 0.0)
    y = jnp.dot(a, w_ref[...], preferred_element_type=jnp.float32) + b_ref[...]
    y_ref[...] = y
    p1 = jnp.sum(y, axis=0, keepdims=True)
    p2 = jnp.sum(y * y, axis=0, keepdims=True)

    @pl.when(pl.program_id(0) == 0)
    def _init():
        s1_ref[...] = p1
        s2_ref[...] = p2

    @pl.when(pl.program_id(0) > 0)
    def _acc():
        s1_ref[...] += p1
        s2_ref[...] += p2


def _mmsub_kernel(x_ref, ctr_ref, w_ref, b_ref, y_ref, s1_ref, s2_ref):
    x = x_ref[...]
    ctr_pad = jnp.pad(ctr_ref[...], ((0, 0), (0, x.shape[1] - 3)))
    y = jnp.dot(x - ctr_pad, w_ref[...],
                preferred_element_type=jnp.float32) + b_ref[...]
    y_ref[...] = y
    p1 = jnp.sum(y, axis=0, keepdims=True)
    p2 = jnp.sum(y * y, axis=0, keepdims=True)

    @pl.when(pl.program_id(0) == 0)
    def _init():
        s1_ref[...] = p1
        s2_ref[...] = p2

    @pl.when(pl.program_id(0) > 0)
    def _acc():
        s1_ref[...] += p1
        s2_ref[...] += p2


def _mmsub(x, ctr, w, b, rblk):
    r, cin = x.shape
    cout = w.shape[1]
    wp = jnp.pad(w, ((0, cin - w.shape[0]), (0, 0)))
    out_specs, out_shape = _stats_out(r, cout, rblk)
    return pl.pallas_call(
        _mmsub_kernel,
        grid=(r // rblk,),
        in_specs=[pl.BlockSpec((rblk, cin), lambda i: (i, 0)),
                  pl.BlockSpec((rblk, 3), lambda i: (i, 0)),
                  pl.BlockSpec((cin, cout), lambda i: (0, 0)),
                  pl.BlockSpec((1, cout), lambda i: (0, 0))],
        out_specs=out_specs,
        out_shape=out_shape,
    )(x, ctr, wp, b.reshape(1, cout))


def _stats_out(r, cout, rblk):
    specs = [pl.BlockSpec((rblk, cout), lambda i: (i, 0)),
             pl.BlockSpec((1, cout), lambda i: (0, 0)),
             pl.BlockSpec((1, cout), lambda i: (0, 0))]
    shapes = [jax.ShapeDtypeStruct((r, cout), jnp.float32),
              jax.ShapeDtypeStruct((1, cout), jnp.float32),
              jax.ShapeDtypeStruct((1, cout), jnp.float32)]
    return specs, shapes


def _mm(x, w, b, rblk):
    r, cin = x.shape
    cout = w.shape[1]
    out_specs, out_shape = _stats_out(r, cout, rblk)
    return pl.pallas_call(
        _mm_kernel,
        grid=(r // rblk,),
        in_specs=[pl.BlockSpec((rblk, cin), lambda i: (i, 0)),
                  pl.BlockSpec((cin, cout), lambda i: (0, 0)),
                  pl.BlockSpec((1, cout), lambda i: (0, 0))],
        out_specs=out_specs,
        out_shape=out_shape,
    )(x, w, b.reshape(1, cout))


def _mmbn(x, s1, s2, g, be, w, b, inv_cnt, rblk):
    r, cin = x.shape
    cout = w.shape[1]
    out_specs, out_shape = _stats_out(r, cout, rblk)
    stat_spec = pl.BlockSpec((1, cin), lambda i: (0, 0))
    return pl.pallas_call(
        functools.partial(_mmbn_kernel, inv_cnt=inv_cnt),
        grid=(r // rblk,),
        in_specs=[pl.BlockSpec((rblk, cin), lambda i: (i, 0)),
                  stat_spec, stat_spec, stat_spec, stat_spec,
                  pl.BlockSpec((cin, cout), lambda i: (0, 0)),
                  pl.BlockSpec((1, cout), lambda i: (0, 0))],
        out_specs=out_specs,
        out_shape=out_shape,
    )(x, s1, s2, g.reshape(1, cin), be.reshape(1, cin), w, b.reshape(1, cout))


def _pool_kernel(y_ref, s1_ref, s2_ref, g_ref, be_ref, o_ref, *, inv_cnt):
    d = y_ref.shape[2]
    m = (s1_ref[...] * inv_cnt).reshape(1, 1, d)
    v = (s2_ref[...] * inv_cnt).reshape(1, 1, d) - m * m
    g = g_ref[...].reshape(1, 1, d)
    be = be_ref[...].reshape(1, 1, d)
    a = jnp.maximum(g * (y_ref[...] - m) / jnp.sqrt(v + 1e-5) + be, 0.0)
    o_ref[...] = jnp.max(a, axis=1)


def _pool(y3, s1, s2, g, be, inv_cnt, gb):
    rows, ns, d = y3.shape
    stat_spec = pl.BlockSpec((1, d), lambda i: (0, 0))
    return pl.pallas_call(
        functools.partial(_pool_kernel, inv_cnt=inv_cnt),
        grid=(rows // gb,),
        in_specs=[pl.BlockSpec((gb, ns, d), lambda i: (i, 0, 0)),
                  stat_spec, stat_spec, stat_spec, stat_spec],
        out_specs=pl.BlockSpec((gb, d), lambda i: (i, 0)),
        out_shape=jax.ShapeDtypeStruct((rows, d), jnp.float32),
        compiler_params=pltpu.CompilerParams(
            dimension_semantics=("parallel",)),
    )(y3, s1, s2, g.reshape(1, d), be.reshape(1, d))


def _sa_msg(xs, ys, zs, pts, npoint, specs, params, rb, lw=128):
    b, n = xs.shape
    cx, cy, cz = _fps_centers(xs, ys, zs, npoint)
    (ra, nsa, _), (rbb, nsb, _) = specs
    ia, ib = _group2i(xs, ys, zs, cx, cy, cz, ra, nsa, rbb, nsb, rb, lw)
    c = pts.shape[-1]
    table = jnp.pad(pts.reshape(b * n, c), ((0, 0), (0, 128 - c)))
    groups = [_sc_gather_rows(table, ia, b * npoint * nsa),
              _sc_gather_rows(table, ib, b * npoint * nsb)]
    ctr3 = jnp.stack([cx, cy, cz], axis=-1).reshape(b * npoint, 1, 3)
    outs = []
    for ((radius, ns, dims), mlp, grouped) in zip(specs, params, groups):
        inv_cnt = 1.0 / (b * npoint * ns)
        w, bb, g, be = mlp[0]
        ctr_rows = jnp.broadcast_to(ctr3, (b * npoint, ns, 3)).reshape(
            b * npoint * ns, 3)
        y, s1, s2 = _mmsub(grouped, ctr_rows, w, bb, rblk=8192)
        for w2, b2, g2, be2 in mlp[1:]:
            y, s1n, s2n = _mmbn(y, s1, s2, g, be, w2, b2, inv_cnt, rblk=8192)
            s1, s2, g, be = s1n, s2n, g2, be2
        pooled = _pool(y.reshape(b * npoint, ns, dims[-1]), s1, s2, g, be,
                       inv_cnt, gb=512)
        outs.append(pooled.reshape(b, npoint, dims[-1]))
    return (cx, cy, cz), jnp.concatenate(outs, axis=-1)


def kernel(pointcloud, params):
    b, n, _ = pointcloud.shape
    xs = pointcloud[..., 0]
    ys = pointcloud[..., 1]
    zs = pointcloud[..., 2]
    (cx1, cy1, cz1), f1 = _sa_msg(xs, ys, zs, pointcloud, _NPOINT1,
                                  _SA1_SPECS, params["sa1"], rb=16)
    pts2 = jnp.concatenate([jnp.stack([cx1, cy1, cz1], axis=-1), f1], axis=-1)
    (cx2, cy2, cz2), f2 = _sa_msg(cx1, cy1, cz1, pts2, _NPOINT2,
                                  _SA2_SPECS, params["sa2"], rb=16)
    lin, _, _ = _mm(f2.reshape(b * _NPOINT2, f2.shape[-1]),
                    params["linear_w"], params["linear_b"], rblk=512)
    xyz2 = jnp.stack([cx2, cy2, cz2], axis=-1)
    return jnp.concatenate([xyz2, lin.reshape(b, _NPOINT2, -1)], axis=-1)


# speedup vs baseline: 1.5162x; 1.0646x over previous
import functools

import jax
import jax.numpy as jnp
from jax import lax
from jax.experimental import pallas as pl
from jax.experimental.pallas import tpu as pltpu
from jax.experimental.pallas import tpu_sc as plsc

_SA1_SPECS = [(0.05, 16, [9, 16, 16, 32]), (0.1, 32, [9, 32, 32, 64])]
_SA2_SPECS = [(0.1, 16, [99, 64, 64, 128]), (0.2, 32, [99, 64, 96, 128])]
_NPOINT1, _NPOINT2 = 1024, 256


def _fps_kernel(xs_ref, ys_ref, zs_ref, cx_ref, cy_ref, cz_ref, dist_ref,
                *, npoint, n):
    b = xs_ref.shape[0]
    xs = xs_ref[...]
    ys = ys_ref[...]
    zs = zs_ref[...]
    col = jax.lax.broadcasted_iota(jnp.int32, (b, n), 1)
    colp = jax.lax.broadcasted_iota(jnp.int32, (b, npoint), 1)
    dist_ref[...] = jnp.full((b, n), 1e10, jnp.float32)
    cx_ref[...] = jnp.zeros((b, npoint), jnp.float32)
    cy_ref[...] = jnp.zeros((b, npoint), jnp.float32)
    cz_ref[...] = jnp.zeros((b, npoint), jnp.float32)

    def body(t, far):
        sel = col == far
        cx = jnp.sum(jnp.where(sel, xs, 0.0), axis=1, keepdims=True)
        cy = jnp.sum(jnp.where(sel, ys, 0.0), axis=1, keepdims=True)
        cz = jnp.sum(jnp.where(sel, zs, 0.0), axis=1, keepdims=True)
        hit = colp == t
        cx_ref[...] = jnp.where(hit, cx, cx_ref[...])
        cy_ref[...] = jnp.where(hit, cy, cy_ref[...])
        cz_ref[...] = jnp.where(hit, cz, cz_ref[...])
        dx = xs - cx
        dy = ys - cy
        dz = zs - cz
        d = dx * dx + dy * dy + dz * dz
        dist = jnp.minimum(dist_ref[...], d)
        dist_ref[...] = dist
        mx = jnp.max(dist, axis=1, keepdims=True)
        far_new = jnp.min(jnp.where(dist == mx, col, n), axis=1, keepdims=True)
        return far_new

    jax.lax.fori_loop(0, npoint, body, jnp.zeros((b, 1), jnp.int32))


def _fps_centers(xs, ys, zs, npoint):
    b, n = xs.shape
    out_shape = [jax.ShapeDtypeStruct((b, npoint), jnp.float32)] * 3
    return pl.pallas_call(
        functools.partial(_fps_kernel, npoint=npoint, n=n),
        out_shape=out_shape,
        scratch_shapes=[pltpu.VMEM((b, n), jnp.float32)],
    )(xs, ys, zs)


def _group2i_kernel(xs_ref, ys_ref, zs_ref, cx_ref, cy_ref, cz_ref,
                    oa_ref, ob_ref, *, r2a, nsa, r2b, nsb, rb, nc, lw, n):
    xs = xs_ref[0].reshape(1, nc, lw)
    ys = ys_ref[0].reshape(1, nc, lw)
    zs = zs_ref[0].reshape(1, nc, lw)
    cxb = cx_ref[0].reshape(rb, 1, 1)
    cyb = cy_ref[0].reshape(rb, 1, 1)
    czb = cz_ref[0].reshape(rb, 1, 1)
    dx = cxb - xs
    dy = cyb - ys
    dz = czb - zs
    sqr = dx * dx + dy * dy + dz * dz

    lio = jax.lax.broadcasted_iota(jnp.int32, (lw, lw), 0)
    ljo = jax.lax.broadcasted_iota(jnp.int32, (lw, lw), 1)
    tri = (lio <= ljo).astype(jnp.float32)
    cio = jax.lax.broadcasted_iota(jnp.int32, (nc, nc), 0)
    cjo = jax.lax.broadcasted_iota(jnp.int32, (nc, nc), 1)
    tri_c = (cio <= cjo).astype(jnp.float32)
    cfio = jax.lax.broadcasted_iota(
        jnp.int32, (rb, 1, nc), 2).astype(jnp.float32)
    lio2 = jax.lax.broadcasted_iota(jnp.int32, (1, lw), 1)
    goff = pl.program_id(0) * n

    for r2, ns, out_ref in ((r2a, nsa, oa_ref), (r2b, nsb, ob_ref)):
        mask2d = (sqr <= r2).reshape(rb * nc, lw)
        mask_f = mask2d.astype(jnp.float32)
        lr = jnp.dot(mask_f, tri, preferred_element_type=jnp.float32)
        lrm = jnp.where(mask2d, lr, 0.0)
        cc = lr[:, lw - 1:lw].reshape(rb, 1, nc)
        ci = jnp.dot(cc.reshape(rb, nc), tri_c,
                     preferred_element_type=jnp.float32).reshape(rb, 1, nc)
        ce = ci - cc
        count = ci[:, :, nc - 1:nc]
        kio = jax.lax.broadcasted_iota(
            jnp.int32, (rb, ns, 1), 1).astype(jnp.float32)
        t = jnp.where(kio < count, kio + 1.0, 1.0)
        oh1 = jnp.logical_and(t > ce, t <= ci).astype(jnp.float32)
        base = jnp.sum(oh1 * ce, axis=2, keepdims=True)
        cidx = jnp.sum(oh1 * cfio, axis=2, keepdims=True)
        lt = (t - base).reshape(rb * ns, 1)
        oh1_2d = oh1.reshape(rb * ns, nc)
        lrg = jnp.concatenate(
            [jnp.dot(oh1_2d[r * ns:(r + 1) * ns, :],
                     lrm[r * nc:(r + 1) * nc, :],
                     preferred_element_type=jnp.float32)
             for r in range(rb)], axis=0)
        lane = jnp.min(jnp.where(lrg == lt, lio2, lw), axis=1,
                       keepdims=True)
        j = cidx.reshape(rb * ns, 1).astype(jnp.int32) * lw + lane + goff
        out_ref[0] = j


def _group2i(xs, ys, zs, cx, cy, cz, ra, nsa, rbb, nsb, rb, lw):
    b, n = xs.shape
    s = cx.shape[1]
    nc = n // lw
    xs3 = xs.reshape(b, nc, lw)
    ys3 = ys.reshape(b, nc, lw)
    zs3 = zs.reshape(b, nc, lw)
    cx3 = cx.reshape(b, s, 1)
    cy3 = cy.reshape(b, s, 1)
    cz3 = cz.reshape(b, s, 1)
    kern = functools.partial(_group2i_kernel, r2a=ra * ra, nsa=nsa,
                             r2b=rbb * rbb, nsb=nsb, rb=rb, nc=nc, lw=lw, n=n)
    row_spec = pl.BlockSpec((1, nc, lw), lambda bi, si: (bi, 0, 0))
    ctr_spec = pl.BlockSpec((1, rb, 1), lambda bi, si: (bi, si, 0))
    return pl.pallas_call(
        kern,
        grid=(b, s // rb),
        in_specs=[row_spec, row_spec, row_spec,
                  ctr_spec, ctr_spec, ctr_spec],
        out_specs=[
            pl.BlockSpec((1, rb * nsa, 1), lambda bi, si: (bi, si, 0)),
            pl.BlockSpec((1, rb * nsb, 1), lambda bi, si: (bi, si, 0)),
        ],
        out_shape=[
            jax.ShapeDtypeStruct((b, s * nsa, 1), jnp.int32),
            jax.ShapeDtypeStruct((b, s * nsb, 1), jnp.int32),
        ],
        compiler_params=pltpu.CompilerParams(
            dimension_semantics=("parallel", "parallel")),
    )(xs3, ys3, zs3, cx3, cy3, cz3)


def _sc_gather(table, idx3, rows):
    d = table.shape[1]
    info = plsc.get_sparse_core_info()
    nw = info.num_cores * info.num_subcores
    b_per_w = rows // nw
    nchunk = b_per_w // 128
    mesh = plsc.VectorSubcoreMesh(core_axis_name="c", subcore_axis_name="s")

    nch_pad = max(nchunk, 8)

    @functools.partial(
        pl.kernel, mesh=mesh,
        out_type=jax.ShapeDtypeStruct((rows, d), jnp.float32),
        scratch_types=[
            pltpu.VMEM((nch_pad, 128), jnp.int32),
            pltpu.VMEM((128, d), jnp.float32),
            pltpu.SemaphoreType.DMA,
        ],
    )
    def k(table_hbm, idx_hbm, out_hbm, idx_v, rows_v, sem):
        wid = lax.axis_index("s") * info.num_cores + lax.axis_index("c")
        base = wid * b_per_w
        pltpu.sync_copy(idx_hbm.at[wid], idx_v)
        for j in range(nchunk):
            pltpu.async_copy(
                table_hbm.at[idx_v.at[j]], rows_v, sem).wait()
            pltpu.sync_copy(rows_v, out_hbm.at[pl.ds(base + j * 128, 128)])

    return k(table, idx3)


def _sc_idx_prep(idx, nw=32):
    flat = idx.reshape(-1)
    nchunk = flat.shape[0] // nw // 128
    i3 = flat.reshape(nw, nchunk, 128)
    if nchunk < 8:
        i3 = jnp.pad(i3, ((0, 0), (0, 8 - nchunk), (0, 0)))
    return i3


def _sc_gather_rows(table, idx, rows, max_chunks=16):
    per_call = 32 * 128 * max_chunks
    if rows <= per_call:
        return _sc_gather(table, _sc_idx_prep(idx), rows)
    flat = idx.reshape(-1)
    parts = [_sc_gather(table, _sc_idx_prep(flat[o:o + per_call]), per_call)
             for o in range(0, rows, per_call)]
    return jnp.concatenate(parts, axis=0)


def _mm_kernel(x_ref, w_ref, b_ref, y_ref, s1_ref, s2_ref):
    y = jnp.dot(x_ref[...], w_ref[...],
                preferred_element_type=jnp.float32) + b_ref[...]
    y_ref[...] = y
    p1 = jnp.sum(y, axis=0, keepdims=True)
    p2 = jnp.sum(y * y, axis=0, keepdims=True)

    @pl.when(pl.program_id(0) == 0)
    def _init():
        s1_ref[...] = p1
        s2_ref[...] = p2

    @pl.when(pl.program_id(0) > 0)
    def _acc():
        s1_ref[...] += p1
        s2_ref[...] += p2


def _mmbn_kernel(x_ref, s1i_ref, s2i_ref, g_ref, be_ref, w_ref, b_ref,
                 y_ref, s1_ref, s2_ref, *, inv_cnt):
    m = s1i_ref[...] * inv_cnt
    v = s2i_ref[...] * inv_cnt - m * m
    a = jnp.maximum(
        g_ref[...] * (x_ref[...] - m) / jnp.sqrt(v + 1e-5) + be_ref[...], 0.0)
    y = jnp.dot(a, w_ref[...], preferred_element_type=jnp.float32) + b_ref[...]
    y_ref[...] = y
    p1 = jnp.sum(y, axis=0, keepdims=True)
    p2 = jnp.sum(y * y, axis=0, keepdims=True)

    @pl.when(pl.program_id(0) == 0)
    def _init():
        s1_ref[...] = p1
        s2_ref[...] = p2

    @pl.when(pl.program_id(0) > 0)
    def _acc():
        s1_ref[...] += p1
        s2_ref[...] += p2


def _mmsub_kernel(x_ref, ctr_ref, w_ref, b_ref, y_ref, s1_ref, s2_ref):
    x = x_ref[...]
    ctr_pad = jnp.pad(ctr_ref[...], ((0, 0), (0, x.shape[1] - 3)))
    y = jnp.dot(x - ctr_pad, w_ref[...],
                preferred_element_type=jnp.float32) + b_ref[...]
    y_ref[...] = y
    p1 = jnp.sum(y, axis=0, keepdims=True)
    p2 = jnp.sum(y * y, axis=0, keepdims=True)

    @pl.when(pl.program_id(0) == 0)
    def _init():
        s1_ref[...] = p1
        s2_ref[...] = p2

    @pl.when(pl.program_id(0) > 0)
    def _acc():
        s1_ref[...] += p1
        s2_ref[...] += p2


def _mmsub(x, ctr, w, b, rblk):
    r, cin = x.shape
    cout = w.shape[1]
    wp = jnp.pad(w, ((0, cin - w.shape[0]), (0, 0)))
    out_specs, out_shape = _stats_out(r, cout, rblk)
    return pl.pallas_call(
        _mmsub_kernel,
        grid=(r // rblk,),
        in_specs=[pl.BlockSpec((rblk, cin), lambda i: (i, 0)),
                  pl.BlockSpec((rblk, 3), lambda i: (i, 0)),
                  pl.BlockSpec((cin, cout), lambda i: (0, 0)),
                  pl.BlockSpec((1, cout), lambda i: (0, 0))],
        out_specs=out_specs,
        out_shape=out_shape,
    )(x, ctr, wp, b.reshape(1, cout))


def _stats_out(r, cout, rblk):
    specs = [pl.BlockSpec((rblk, cout), lambda i: (i, 0)),
             pl.BlockSpec((1, cout), lambda i: (0, 0)),
             pl.BlockSpec((1, cout), lambda i: (0, 0))]
    shapes = [jax.ShapeDtypeStruct((r, cout), jnp.float32),
              jax.ShapeDtypeStruct((1, cout), jnp.float32),
              jax.ShapeDtypeStruct((1, cout), jnp.float32)]
    return specs, shapes


def _mm(x, w, b, rblk):
    r, cin = x.shape
    cout = w.shape[1]
    out_specs, out_shape = _stats_out(r, cout, rblk)
    return pl.pallas_call(
        _mm_kernel,
        grid=(r // rblk,),
        in_specs=[pl.BlockSpec((rblk, cin), lambda i: (i, 0)),
                  pl.BlockSpec((cin, cout), lambda i: (0, 0)),
                  pl.BlockSpec((1, cout), lambda i: (0, 0))],
        out_specs=out_specs,
        out_shape=out_shape,
    )(x, w, b.reshape(1, cout))


def _mmbn(x, s1, s2, g, be, w, b, inv_cnt, rblk):
    r, cin = x.shape
    cout = w.shape[1]
    out_specs, out_shape = _stats_out(r, cout, rblk)
    stat_spec = pl.BlockSpec((1, cin), lambda i: (0, 0))
    return pl.pallas_call(
        functools.partial(_mmbn_kernel, inv_cnt=inv_cnt),
        grid=(r // rblk,),
        in_specs=[pl.BlockSpec((rblk, cin), lambda i: (i, 0)),
                  stat_spec, stat_spec, stat_spec, stat_spec,
                  pl.BlockSpec((cin, cout), lambda i: (0, 0)),
                  pl.BlockSpec((1, cout), lambda i: (0, 0))],
        out_specs=out_specs,
        out_shape=out_shape,
    )(x, s1, s2, g.reshape(1, cin), be.reshape(1, cin), w, b.reshape(1, cout))


def _pool_kernel(y_ref, s1_ref, s2_ref, g_ref, be_ref, o_ref, *, inv_cnt):
    d = y_ref.shape[2]
    m = (s1_ref[...] * inv_cnt).reshape(1, 1, d)
    v = (s2_ref[...] * inv_cnt).reshape(1, 1, d) - m * m
    g = g_ref[...].reshape(1, 1, d)
    be = be_ref[...].reshape(1, 1, d)
    a = jnp.maximum(g * (y_ref[...] - m) / jnp.sqrt(v + 1e-5) + be, 0.0)
    o_ref[...] = jnp.max(a, axis=1)


def _pool(y3, s1, s2, g, be, inv_cnt, gb):
    rows, ns, d = y3.shape
    stat_spec = pl.BlockSpec((1, d), lambda i: (0, 0))
    return pl.pallas_call(
        functools.partial(_pool_kernel, inv_cnt=inv_cnt),
        grid=(rows // gb,),
        in_specs=[pl.BlockSpec((gb, ns, d), lambda i: (i, 0, 0)),
                  stat_spec, stat_spec, stat_spec, stat_spec],
        out_specs=pl.BlockSpec((gb, d), lambda i: (i, 0)),
        out_shape=jax.ShapeDtypeStruct((rows, d), jnp.float32),
        compiler_params=pltpu.CompilerParams(
            dimension_semantics=("parallel",)),
    )(y3, s1, s2, g.reshape(1, d), be.reshape(1, d))


def _sa_msg(xs, ys, zs, pts, npoint, specs, params, rb, lw=128):
    b, n = xs.shape
    cx, cy, cz = _fps_centers(xs, ys, zs, npoint)
    (ra, nsa, _), (rbb, nsb, _) = specs
    ia, ib = _group2i(xs, ys, zs, cx, cy, cz, ra, nsa, rbb, nsb, rb, lw)
    c = pts.shape[-1]
    table = jnp.pad(pts.reshape(b * n, c), ((0, 0), (0, 128 - c)))
    groups = [_sc_gather_rows(table, ia, b * npoint * nsa),
              _sc_gather_rows(table, ib, b * npoint * nsb)]
    ctr3 = jnp.stack([cx, cy, cz], axis=-1).reshape(b * npoint, 1, 3)
    outs = []
    for ((radius, ns, dims), mlp, grouped) in zip(specs, params, groups):
        inv_cnt = 1.0 / (b * npoint * ns)
        w, bb, g, be = mlp[0]
        ctr_rows = jnp.broadcast_to(ctr3, (b * npoint, ns, 3)).reshape(
            b * npoint * ns, 3)
        y, s1, s2 = _mmsub(grouped, ctr_rows, w, bb, rblk=8192)
        for w2, b2, g2, be2 in mlp[1:]:
            y, s1n, s2n = _mmbn(y, s1, s2, g, be, w2, b2, inv_cnt, rblk=8192)
            s1, s2, g, be = s1n, s2n, g2, be2
        pooled = _pool(y.reshape(b * npoint, ns, dims[-1]), s1, s2, g, be,
                       inv_cnt, gb=512)
        outs.append(pooled.reshape(b, npoint, dims[-1]))
    return (cx, cy, cz), jnp.concatenate(outs, axis=-1)


def kernel(pointcloud, params):
    b, n, _ = pointcloud.shape
    xs = pointcloud[..., 0]
    ys = pointcloud[..., 1]
    zs = pointcloud[..., 2]
    (cx1, cy1, cz1), f1 = _sa_msg(xs, ys, zs, pointcloud, _NPOINT1,
                                  _SA1_SPECS, params["sa1"], rb=16)
    pts2 = jnp.concatenate([jnp.stack([cx1, cy1, cz1], axis=-1), f1], axis=-1)
    (cx2, cy2, cz2), f2 = _sa_msg(cx1, cy1, cz1, pts2, _NPOINT2,
                                  _SA2_SPECS, params["sa2"], rb=16)
    lin, _, _ = _mm(f2.reshape(b * _NPOINT2, f2.shape[-1]),
                    params["linear_w"], params["linear_b"], rblk=512)
    xyz2 = jnp.stack([cx2, cy2, cz2], axis=-1)
    return jnp.concatenate([xyz2, lin.reshape(b, _NPOINT2, -1)], axis=-1)


# speedup vs baseline: 1.6153x; 1.0653x over previous
import functools

import jax
import jax.numpy as jnp
from jax import lax
from jax.experimental import pallas as pl
from jax.experimental.pallas import tpu as pltpu
from jax.experimental.pallas import tpu_sc as plsc

_SA1_SPECS = [(0.05, 16, [9, 16, 16, 32]), (0.1, 32, [9, 32, 32, 64])]
_SA2_SPECS = [(0.1, 16, [99, 64, 64, 128]), (0.2, 32, [99, 64, 96, 128])]
_NPOINT1, _NPOINT2 = 1024, 256


def _fps_kernel(xs_ref, ys_ref, zs_ref, cx_ref, cy_ref, cz_ref, dist_ref,
                *, npoint, n):
    b = xs_ref.shape[0]
    xs = xs_ref[...]
    ys = ys_ref[...]
    zs = zs_ref[...]
    col = jax.lax.broadcasted_iota(jnp.int32, (b, n), 1)
    colp = jax.lax.broadcasted_iota(jnp.int32, (b, npoint), 1)
    dist_ref[...] = jnp.full((b, n), 1e10, jnp.float32)
    cx_ref[...] = jnp.zeros((b, npoint), jnp.float32)
    cy_ref[...] = jnp.zeros((b, npoint), jnp.float32)
    cz_ref[...] = jnp.zeros((b, npoint), jnp.float32)

    def body(t, far):
        sel = col == far
        cx = jnp.sum(jnp.where(sel, xs, 0.0), axis=1, keepdims=True)
        cy = jnp.sum(jnp.where(sel, ys, 0.0), axis=1, keepdims=True)
        cz = jnp.sum(jnp.where(sel, zs, 0.0), axis=1, keepdims=True)
        hit = colp == t
        cx_ref[...] = jnp.where(hit, cx, cx_ref[...])
        cy_ref[...] = jnp.where(hit, cy, cy_ref[...])
        cz_ref[...] = jnp.where(hit, cz, cz_ref[...])
        dx = xs - cx
        dy = ys - cy
        dz = zs - cz
        d = dx * dx + dy * dy + dz * dz
        dist = jnp.minimum(dist_ref[...], d)
        dist_ref[...] = dist
        mx = jnp.max(dist, axis=1, keepdims=True)
        far_new = jnp.min(jnp.where(dist == mx, col, n), axis=1, keepdims=True)
        return far_new

    jax.lax.fori_loop(0, npoint, body, jnp.zeros((b, 1), jnp.int32))


def _fps_centers(xs, ys, zs, npoint):
    b, n = xs.shape
    out_shape = [jax.ShapeDtypeStruct((b, npoint), jnp.float32)] * 3
    return pl.pallas_call(
        functools.partial(_fps_kernel, npoint=npoint, n=n),
        out_shape=out_shape,
        scratch_shapes=[pltpu.VMEM((b, n), jnp.float32)],
    )(xs, ys, zs)


def _group2i_kernel(xs_ref, ys_ref, zs_ref, cx_ref, cy_ref, cz_ref,
                    oa_ref, ob_ref, *, r2a, nsa, r2b, nsb, rb, nc, lw, n):
    xs = xs_ref[0].reshape(1, nc, lw)
    ys = ys_ref[0].reshape(1, nc, lw)
    zs = zs_ref[0].reshape(1, nc, lw)
    cxb = cx_ref[0].reshape(rb, 1, 1)
    cyb = cy_ref[0].reshape(rb, 1, 1)
    czb = cz_ref[0].reshape(rb, 1, 1)
    dx = cxb - xs
    dy = cyb - ys
    dz = czb - zs
    sqr = dx * dx + dy * dy + dz * dz

    lio = jax.lax.broadcasted_iota(jnp.int32, (lw, lw), 0)
    ljo = jax.lax.broadcasted_iota(jnp.int32, (lw, lw), 1)
    tri = (lio <= ljo).astype(jnp.float32)
    cio = jax.lax.broadcasted_iota(jnp.int32, (nc, nc), 0)
    cjo = jax.lax.broadcasted_iota(jnp.int32, (nc, nc), 1)
    tri_c = (cio <= cjo).astype(jnp.float32)
    cfio = jax.lax.broadcasted_iota(
        jnp.int32, (rb, 1, nc), 2).astype(jnp.float32)
    lio2 = jax.lax.broadcasted_iota(jnp.int32, (1, lw), 1)
    goff = pl.program_id(0) * n

    for r2, ns, out_ref in ((r2a, nsa, oa_ref), (r2b, nsb, ob_ref)):
        mask2d = (sqr <= r2).reshape(rb * nc, lw)
        mask_f = mask2d.astype(jnp.float32)
        lr = jnp.dot(mask_f, tri, preferred_element_type=jnp.float32)
        lrm = jnp.where(mask2d, lr, 0.0)
        cc = lr[:, lw - 1:lw].reshape(rb, 1, nc)
        ci = jnp.dot(cc.reshape(rb, nc), tri_c,
                     preferred_element_type=jnp.float32).reshape(rb, 1, nc)
        ce = ci - cc
        count = ci[:, :, nc - 1:nc]
        kio = jax.lax.broadcasted_iota(
            jnp.int32, (rb, ns, 1), 1).astype(jnp.float32)
        t = jnp.where(kio < count, kio + 1.0, 1.0)
        oh1 = jnp.logical_and(t > ce, t <= ci).astype(jnp.float32)
        base = jnp.sum(oh1 * ce, axis=2, keepdims=True)
        cidx = jnp.sum(oh1 * cfio, axis=2, keepdims=True)
        lt = (t - base).reshape(rb * ns, 1)
        oh1_2d = oh1.reshape(rb * ns, nc)
        lrg = jnp.concatenate(
            [jnp.dot(oh1_2d[r * ns:(r + 1) * ns, :],
                     lrm[r * nc:(r + 1) * nc, :],
                     preferred_element_type=jnp.float32)
             for r in range(rb)], axis=0)
        lane = jnp.min(jnp.where(lrg == lt, lio2, lw), axis=1,
                       keepdims=True)
        j = cidx.reshape(rb * ns, 1).astype(jnp.int32) * lw + lane + goff
        out_ref[0] = j


def _group2i(xs, ys, zs, cx, cy, cz, ra, nsa, rbb, nsb, rb, lw):
    b, n = xs.shape
    s = cx.shape[1]
    nc = n // lw
    xs3 = xs.reshape(b, nc, lw)
    ys3 = ys.reshape(b, nc, lw)
    zs3 = zs.reshape(b, nc, lw)
    cx3 = cx.reshape(b, s, 1)
    cy3 = cy.reshape(b, s, 1)
    cz3 = cz.reshape(b, s, 1)
    kern = functools.partial(_group2i_kernel, r2a=ra * ra, nsa=nsa,
                             r2b=rbb * rbb, nsb=nsb, rb=rb, nc=nc, lw=lw, n=n)
    row_spec = pl.BlockSpec((1, nc, lw), lambda bi, si: (bi, 0, 0))
    ctr_spec = pl.BlockSpec((1, rb, 1), lambda bi, si: (bi, si, 0))
    return pl.pallas_call(
        kern,
        grid=(b, s // rb),
        in_specs=[row_spec, row_spec, row_spec,
                  ctr_spec, ctr_spec, ctr_spec],
        out_specs=[
            pl.BlockSpec((1, rb * nsa, 1), lambda bi, si: (bi, si, 0)),
            pl.BlockSpec((1, rb * nsb, 1), lambda bi, si: (bi, si, 0)),
        ],
        out_shape=[
            jax.ShapeDtypeStruct((b, s * nsa, 1), jnp.int32),
            jax.ShapeDtypeStruct((b, s * nsb, 1), jnp.int32),
        ],
        compiler_params=pltpu.CompilerParams(
            dimension_semantics=("parallel", "parallel")),
    )(xs3, ys3, zs3, cx3, cy3, cz3)


def _sc_gather(table, idx3, rows):
    d = table.shape[1]
    info = plsc.get_sparse_core_info()
    nw = info.num_cores * info.num_subcores
    b_per_w = rows // nw
    nchunk = b_per_w // 128
    mesh = plsc.VectorSubcoreMesh(core_axis_name="c", subcore_axis_name="s")

    nch_pad = max(nchunk, 8)

    @functools.partial(
        pl.kernel, mesh=mesh,
        out_type=jax.ShapeDtypeStruct((rows, d), jnp.float32),
        scratch_types=[
            pltpu.VMEM((nch_pad, 128), jnp.int32),
            pltpu.VMEM((128, d), jnp.float32),
            pltpu.SemaphoreType.DMA,
        ],
    )
    def k(table_hbm, idx_hbm, out_hbm, idx_v, rows_v, sem):
        wid = lax.axis_index("s") * info.num_cores + lax.axis_index("c")
        base = wid * b_per_w
        pltpu.sync_copy(idx_hbm.at[wid], idx_v)
        for j in range(nchunk):
            pltpu.async_copy(
                table_hbm.at[idx_v.at[j]], rows_v, sem).wait()
            pltpu.sync_copy(rows_v, out_hbm.at[pl.ds(base + j * 128, 128)])

    return k(table, idx3)


def _sc_idx_prep(idx, nw=32):
    flat = idx.reshape(-1)
    nchunk = flat.shape[0] // nw // 128
    i3 = flat.reshape(nw, nchunk, 128)
    if nchunk < 8:
        i3 = jnp.pad(i3, ((0, 0), (0, 8 - nchunk), (0, 0)))
    return i3


def _sc_gather_rows(table, idx, rows, max_chunks=16):
    per_call = 32 * 128 * max_chunks
    if rows <= per_call:
        return _sc_gather(table, _sc_idx_prep(idx), rows)
    flat = idx.reshape(-1)
    parts = [_sc_gather(table, _sc_idx_prep(flat[o:o + per_call]), per_call)
             for o in range(0, rows, per_call)]
    return jnp.concatenate(parts, axis=0)


def _mm_kernel(x_ref, w_ref, b_ref, y_ref, s1_ref, s2_ref):
    y = jnp.dot(x_ref[...], w_ref[...],
                preferred_element_type=jnp.float32) + b_ref[...]
    y_ref[...] = y
    p1 = jnp.sum(y, axis=0, keepdims=True)
    p2 = jnp.sum(y * y, axis=0, keepdims=True)

    @pl.when(pl.program_id(0) == 0)
    def _init():
        s1_ref[...] = p1
        s2_ref[...] = p2

    @pl.when(pl.program_id(0) > 0)
    def _acc():
        s1_ref[...] += p1
        s2_ref[...] += p2


def _mmbn_kernel(x_ref, s1i_ref, s2i_ref, g_ref, be_ref, w_ref, b_ref,
                 y_ref, s1_ref, s2_ref, *, inv_cnt):
    m = s1i_ref[...] * inv_cnt
    v = s2i_ref[...] * inv_cnt - m * m
    a = jnp.maximum(
        g_ref[...] * (x_ref[...] - m) / jnp.sqrt(v + 1e-5) + be_ref[...], 0.0)
    y = jnp.dot(a, w_ref[...], preferred_element_type=jnp.float32) + b_ref[...]
    y_ref[...] = y
    p1 = jnp.sum(y, axis=0, keepdims=True)
    p2 = jnp.sum(y * y, axis=0, keepdims=True)

    @pl.when(pl.program_id(0) == 0)
    def _init():
        s1_ref[...] = p1
        s2_ref[...] = p2

    @pl.when(pl.program_id(0) > 0)
    def _acc():
        s1_ref[...] += p1
        s2_ref[...] += p2


def _mmsub_kernel(x_ref, ctr_ref, w_ref, b_ref, y_ref, s1_ref, s2_ref):
    x = x_ref[...]
    ctr_pad = jnp.pad(ctr_ref[...], ((0, 0), (0, x.shape[1] - 3)))
    y = jnp.dot(x - ctr_pad, w_ref[...],
                preferred_element_type=jnp.float32) + b_ref[...]
    y_ref[...] = y
    p1 = jnp.sum(y, axis=0, keepdims=True)
    p2 = jnp.sum(y * y, axis=0, keepdims=True)

    @pl.when(pl.program_id(0) == 0)
    def _init():
        s1_ref[...] = p1
        s2_ref[...] = p2

    @pl.when(pl.program_id(0) > 0)
    def _acc():
        s1_ref[...] += p1
        s2_ref[...] += p2


def _mmsub(x, ctr, w, b, rblk):
    r, cin = x.shape
    cout = w.shape[1]
    wp = jnp.pad(w, ((0, cin - w.shape[0]), (0, 0)))
    out_specs, out_shape = _stats_out(r, cout, rblk)
    return pl.pallas_call(
        _mmsub_kernel,
        grid=(r // rblk,),
        in_specs=[pl.BlockSpec((rblk, cin), lambda i: (i, 0)),
                  pl.BlockSpec((rblk, 3), lambda i: (i, 0)),
                  pl.BlockSpec((cin, cout), lambda i: (0, 0)),
                  pl.BlockSpec((1, cout), lambda i: (0, 0))],
        out_specs=out_specs,
        out_shape=out_shape,
    )(x, ctr, wp, b.reshape(1, cout))


def _stats_out(r, cout, rblk):
    specs = [pl.BlockSpec((rblk, cout), lambda i: (i, 0)),
             pl.BlockSpec((1, cout), lambda i: (0, 0)),
             pl.BlockSpec((1, cout), lambda i: (0, 0))]
    shapes = [jax.ShapeDtypeStruct((r, cout), jnp.float32),
              jax.ShapeDtypeStruct((1, cout), jnp.float32),
              jax.ShapeDtypeStruct((1, cout), jnp.float32)]
    return specs, shapes


def _mm(x, w, b, rblk):
    r, cin = x.shape
    cout = w.shape[1]
    out_specs, out_shape = _stats_out(r, cout, rblk)
    return pl.pallas_call(
        _mm_kernel,
        grid=(r // rblk,),
        in_specs=[pl.BlockSpec((rblk, cin), lambda i: (i, 0)),
                  pl.BlockSpec((cin, cout), lambda i: (0, 0)),
                  pl.BlockSpec((1, cout), lambda i: (0, 0))],
        out_specs=out_specs,
        out_shape=out_shape,
    )(x, w, b.reshape(1, cout))


def _mmbn(x, s1, s2, g, be, w, b, inv_cnt, rblk):
    r, cin = x.shape
    cout = w.shape[1]
    out_specs, out_shape = _stats_out(r, cout, rblk)
    stat_spec = pl.BlockSpec((1, cin), lambda i: (0, 0))
    return pl.pallas_call(
        functools.partial(_mmbn_kernel, inv_cnt=inv_cnt),
        grid=(r // rblk,),
        in_specs=[pl.BlockSpec((rblk, cin), lambda i: (i, 0)),
                  stat_spec, stat_spec, stat_spec, stat_spec,
                  pl.BlockSpec((cin, cout), lambda i: (0, 0)),
                  pl.BlockSpec((1, cout), lambda i: (0, 0))],
        out_specs=out_specs,
        out_shape=out_shape,
    )(x, s1, s2, g.reshape(1, cin), be.reshape(1, cin), w, b.reshape(1, cout))


def _pool_kernel(y_ref, s1_ref, s2_ref, g_ref, be_ref, o_ref, *, inv_cnt):
    d = y_ref.shape[2]
    m = (s1_ref[...] * inv_cnt).reshape(1, 1, d)
    v = (s2_ref[...] * inv_cnt).reshape(1, 1, d) - m * m
    g = g_ref[...].reshape(1, 1, d)
    be = be_ref[...].reshape(1, 1, d)
    a = jnp.maximum(g * (y_ref[...] - m) / jnp.sqrt(v + 1e-5) + be, 0.0)
    o_ref[...] = jnp.max(a, axis=1)


def _pool(y3, s1, s2, g, be, inv_cnt, gb):
    rows, ns, d = y3.shape
    stat_spec = pl.BlockSpec((1, d), lambda i: (0, 0))
    return pl.pallas_call(
        functools.partial(_pool_kernel, inv_cnt=inv_cnt),
        grid=(rows // gb,),
        in_specs=[pl.BlockSpec((gb, ns, d), lambda i: (i, 0, 0)),
                  stat_spec, stat_spec, stat_spec, stat_spec],
        out_specs=pl.BlockSpec((gb, d), lambda i: (i, 0)),
        out_shape=jax.ShapeDtypeStruct((rows, d), jnp.float32),
        compiler_params=pltpu.CompilerParams(
            dimension_semantics=("parallel",)),
    )(y3, s1, s2, g.reshape(1, d), be.reshape(1, d))


def _sa_msg(xs, ys, zs, pts, npoint, specs, params, rb, lw=128):
    b, n = xs.shape
    cx, cy, cz = _fps_centers(xs, ys, zs, npoint)
    (ra, nsa, _), (rbb, nsb, _) = specs
    ia, ib = _group2i(xs, ys, zs, cx, cy, cz, ra, nsa, rbb, nsb, rb, lw)
    c = pts.shape[-1]
    table = jnp.pad(pts.reshape(b * n, c), ((0, 0), (0, 128 - c)))
    groups = [_sc_gather_rows(table, ia, b * npoint * nsa),
              _sc_gather_rows(table, ib, b * npoint * nsb)]
    ctr3 = jnp.stack([cx, cy, cz], axis=-1).reshape(b * npoint, 1, 3)
    outs = []
    for ((radius, ns, dims), mlp, grouped) in zip(specs, params, groups):
        inv_cnt = 1.0 / (b * npoint * ns)
        w, bb, g, be = mlp[0]
        ctr_rows = jnp.broadcast_to(ctr3, (b * npoint, ns, 3)).reshape(
            b * npoint * ns, 3)
        y, s1, s2 = _mmsub(grouped, ctr_rows, w, bb, rblk=8192)
        for w2, b2, g2, be2 in mlp[1:]:
            y, s1n, s2n = _mmbn(y, s1, s2, g, be, w2, b2, inv_cnt, rblk=8192)
            s1, s2, g, be = s1n, s2n, g2, be2
        pooled = _pool(y.reshape(b * npoint, ns, dims[-1]), s1, s2, g, be,
                       inv_cnt, gb=512)
        outs.append(pooled.reshape(b, npoint, dims[-1]))
    return (cx, cy, cz), jnp.concatenate(outs, axis=-1)


def kernel(pointcloud, params):
    b, n, _ = pointcloud.shape
    xs = pointcloud[..., 0]
    ys = pointcloud[..., 1]
    zs = pointcloud[..., 2]
    (cx1, cy1, cz1), f1 = _sa_msg(xs, ys, zs, pointcloud, _NPOINT1,
                                  _SA1_SPECS, params["sa1"], rb=32)
    pts2 = jnp.concatenate([jnp.stack([cx1, cy1, cz1], axis=-1), f1], axis=-1)
    (cx2, cy2, cz2), f2 = _sa_msg(cx1, cy1, cz1, pts2, _NPOINT2,
                                  _SA2_SPECS, params["sa2"], rb=32)
    lin, _, _ = _mm(f2.reshape(b * _NPOINT2, f2.shape[-1]),
                    params["linear_w"], params["linear_b"], rblk=512)
    xyz2 = jnp.stack([cx2, cy2, cz2], axis=-1)
    return jnp.concatenate([xyz2, lin.reshape(b, _NPOINT2, -1)], axis=-1)


# speedup vs baseline: 1.6405x; 1.0156x over previous
import functools

import jax
import jax.numpy as jnp
from jax import lax
from jax.experimental import pallas as pl
from jax.experimental.pallas import tpu as pltpu
from jax.experimental.pallas import tpu_sc as plsc

_SA1_SPECS = [(0.05, 16, [9, 16, 16, 32]), (0.1, 32, [9, 32, 32, 64])]
_SA2_SPECS = [(0.1, 16, [99, 64, 64, 128]), (0.2, 32, [99, 64, 96, 128])]
_NPOINT1, _NPOINT2 = 1024, 256


def _fps_kernel(xs_ref, ys_ref, zs_ref, cx_ref, cy_ref, cz_ref, dist_ref,
                *, npoint, n):
    b = xs_ref.shape[0]
    xs = xs_ref[...]
    ys = ys_ref[...]
    zs = zs_ref[...]
    col = jax.lax.broadcasted_iota(jnp.int32, (b, n), 1)
    colp = jax.lax.broadcasted_iota(jnp.int32, (b, npoint), 1)
    dist_ref[...] = jnp.full((b, n), 1e10, jnp.float32)
    cx_ref[...] = jnp.zeros((b, npoint), jnp.float32)
    cy_ref[...] = jnp.zeros((b, npoint), jnp.float32)
    cz_ref[...] = jnp.zeros((b, npoint), jnp.float32)

    def body(t, far):
        sel = col == far
        cx = jnp.sum(jnp.where(sel, xs, 0.0), axis=1, keepdims=True)
        cy = jnp.sum(jnp.where(sel, ys, 0.0), axis=1, keepdims=True)
        cz = jnp.sum(jnp.where(sel, zs, 0.0), axis=1, keepdims=True)
        hit = colp == t
        cx_ref[...] = jnp.where(hit, cx, cx_ref[...])
        cy_ref[...] = jnp.where(hit, cy, cy_ref[...])
        cz_ref[...] = jnp.where(hit, cz, cz_ref[...])
        dx = xs - cx
        dy = ys - cy
        dz = zs - cz
        d = dx * dx + dy * dy + dz * dz
        dist = jnp.minimum(dist_ref[...], d)
        dist_ref[...] = dist
        mx = jnp.max(dist, axis=1, keepdims=True)
        far_new = jnp.min(jnp.where(dist == mx, col, n), axis=1, keepdims=True)
        return far_new

    jax.lax.fori_loop(0, npoint, body, jnp.zeros((b, 1), jnp.int32))


def _fps_centers(xs, ys, zs, npoint):
    b, n = xs.shape
    out_shape = [jax.ShapeDtypeStruct((b, npoint), jnp.float32)] * 3
    return pl.pallas_call(
        functools.partial(_fps_kernel, npoint=npoint, n=n),
        out_shape=out_shape,
        scratch_shapes=[pltpu.VMEM((b, n), jnp.float32)],
    )(xs, ys, zs)


def _group2i_kernel(xs_ref, ys_ref, zs_ref, cx_ref, cy_ref, cz_ref,
                    oa_ref, ob_ref, *, r2a, nsa, r2b, nsb, rb, nc, lw, n):
    xs = xs_ref[0].reshape(1, nc, lw)
    ys = ys_ref[0].reshape(1, nc, lw)
    zs = zs_ref[0].reshape(1, nc, lw)
    cxb = cx_ref[0].reshape(rb, 1, 1)
    cyb = cy_ref[0].reshape(rb, 1, 1)
    czb = cz_ref[0].reshape(rb, 1, 1)
    dx = cxb - xs
    dy = cyb - ys
    dz = czb - zs
    sqr = dx * dx + dy * dy + dz * dz

    lio = jax.lax.broadcasted_iota(jnp.int32, (lw, lw), 0)
    ljo = jax.lax.broadcasted_iota(jnp.int32, (lw, lw), 1)
    tri = (lio <= ljo).astype(jnp.float32)
    cio = jax.lax.broadcasted_iota(jnp.int32, (nc, nc), 0)
    cjo = jax.lax.broadcasted_iota(jnp.int32, (nc, nc), 1)
    tri_c = (cio <= cjo).astype(jnp.float32)
    cfio = jax.lax.broadcasted_iota(
        jnp.int32, (rb, 1, nc), 2).astype(jnp.float32)
    lio2 = jax.lax.broadcasted_iota(jnp.int32, (1, lw), 1)
    goff = pl.program_id(0) * n

    for r2, ns, out_ref in ((r2a, nsa, oa_ref), (r2b, nsb, ob_ref)):
        mask2d = (sqr <= r2).reshape(rb * nc, lw)
        mask_f = mask2d.astype(jnp.float32)
        lr = jnp.dot(mask_f, tri, preferred_element_type=jnp.float32)
        lrm = jnp.where(mask2d, lr, 0.0)
        cc = lr[:, lw - 1:lw].reshape(rb, 1, nc)
        ci = jnp.dot(cc.reshape(rb, nc), tri_c,
                     preferred_element_type=jnp.float32).reshape(rb, 1, nc)
        ce = ci - cc
        count = ci[:, :, nc - 1:nc]
        kio = jax.lax.broadcasted_iota(
            jnp.int32, (rb, ns, 1), 1).astype(jnp.float32)
        t = jnp.where(kio < count, kio + 1.0, 1.0)
        oh1 = jnp.logical_and(t > ce, t <= ci).astype(jnp.float32)
        base = jnp.sum(oh1 * ce, axis=2, keepdims=True)
        cidx = jnp.sum(oh1 * cfio, axis=2, keepdims=True)
        lt = (t - base).reshape(rb * ns, 1)
        oh1_2d = oh1.reshape(rb * ns, nc)
        lrg = jnp.concatenate(
            [jnp.dot(oh1_2d[r * ns:(r + 1) * ns, :],
                     lrm[r * nc:(r + 1) * nc, :],
                     preferred_element_type=jnp.float32)
             for r in range(rb)], axis=0)
        lane = jnp.min(jnp.where(lrg == lt, lio2, lw), axis=1,
                       keepdims=True)
        j = cidx.reshape(rb * ns, 1).astype(jnp.int32) * lw + lane + goff
        out_ref[0] = j


def _group2i(xs, ys, zs, cx, cy, cz, ra, nsa, rbb, nsb, rb, lw):
    b, n = xs.shape
    s = cx.shape[1]
    nc = n // lw
    xs3 = xs.reshape(b, nc, lw)
    ys3 = ys.reshape(b, nc, lw)
    zs3 = zs.reshape(b, nc, lw)
    cx3 = cx.reshape(b, s, 1)
    cy3 = cy.reshape(b, s, 1)
    cz3 = cz.reshape(b, s, 1)
    kern = functools.partial(_group2i_kernel, r2a=ra * ra, nsa=nsa,
                             r2b=rbb * rbb, nsb=nsb, rb=rb, nc=nc, lw=lw, n=n)
    row_spec = pl.BlockSpec((1, nc, lw), lambda bi, si: (bi, 0, 0))
    ctr_spec = pl.BlockSpec((1, rb, 1), lambda bi, si: (bi, si, 0))
    return pl.pallas_call(
        kern,
        grid=(b, s // rb),
        in_specs=[row_spec, row_spec, row_spec,
                  ctr_spec, ctr_spec, ctr_spec],
        out_specs=[
            pl.BlockSpec((1, rb * nsa, 1), lambda bi, si: (bi, si, 0)),
            pl.BlockSpec((1, rb * nsb, 1), lambda bi, si: (bi, si, 0)),
        ],
        out_shape=[
            jax.ShapeDtypeStruct((b, s * nsa, 1), jnp.int32),
            jax.ShapeDtypeStruct((b, s * nsb, 1), jnp.int32),
        ],
        compiler_params=pltpu.CompilerParams(
            dimension_semantics=("parallel", "parallel")),
    )(xs3, ys3, zs3, cx3, cy3, cz3)


def _sc_gather(table, idx3, rows):
    d = table.shape[1]
    info = plsc.get_sparse_core_info()
    nw = info.num_cores * info.num_subcores
    b_per_w = rows // nw
    nchunk = b_per_w // 128
    mesh = plsc.VectorSubcoreMesh(core_axis_name="c", subcore_axis_name="s")

    nch_pad = max(nchunk, 8)

    @functools.partial(
        pl.kernel, mesh=mesh,
        out_type=jax.ShapeDtypeStruct((rows, d), jnp.float32),
        scratch_types=[
            pltpu.VMEM((nch_pad, 128), jnp.int32),
            pltpu.VMEM((128, d), jnp.float32),
            pltpu.SemaphoreType.DMA,
        ],
    )
    def k(table_hbm, idx_hbm, out_hbm, idx_v, rows_v, sem):
        wid = lax.axis_index("s") * info.num_cores + lax.axis_index("c")
        base = wid * b_per_w
        pltpu.sync_copy(idx_hbm.at[wid], idx_v)
        for j in range(nchunk):
            pltpu.async_copy(
                table_hbm.at[idx_v.at[j]], rows_v, sem).wait()
            pltpu.sync_copy(rows_v, out_hbm.at[pl.ds(base + j * 128, 128)])

    return k(table, idx3)


def _sc_idx_prep(idx, nw=32):
    flat = idx.reshape(-1)
    nchunk = flat.shape[0] // nw // 128
    i3 = flat.reshape(nw, nchunk, 128)
    if nchunk < 8:
        i3 = jnp.pad(i3, ((0, 0), (0, 8 - nchunk), (0, 0)))
    return i3


def _sc_gather_rows(table, idx, rows, max_chunks=16):
    per_call = 32 * 128 * max_chunks
    if rows <= per_call:
        return _sc_gather(table, _sc_idx_prep(idx), rows)
    flat = idx.reshape(-1)
    parts = [_sc_gather(table, _sc_idx_prep(flat[o:o + per_call]), per_call)
             for o in range(0, rows, per_call)]
    return jnp.concatenate(parts, axis=0)


def _mm_kernel(x_ref, w_ref, b_ref, y_ref, s1_ref, s2_ref):
    y = jnp.dot(x_ref[...], w_ref[...],
                preferred_element_type=jnp.float32) + b_ref[...]
    y_ref[...] = y
    p1 = jnp.sum(y, axis=0, keepdims=True)
    p2 = jnp.sum(y * y, axis=0, keepdims=True)

    @pl.when(pl.program_id(0) == 0)
    def _init():
        s1_ref[...] = p1
        s2_ref[...] = p2

    @pl.when(pl.program_id(0) > 0)
    def _acc():
        s1_ref[...] += p1
        s2_ref[...] += p2


def _mmbn_kernel(x_ref, s1i_ref, s2i_ref, g_ref, be_ref, w_ref, b_ref,
                 y_ref, s1_ref, s2_ref, *, inv_cnt):
    m = s1i_ref[...] * inv_cnt
    v = s2i_ref[...] * inv_cnt - m * m
    a = jnp.maximum(
        g_ref[...] * (x_ref[...] - m) / jnp.sqrt(v + 1e-5) + be_ref[...], 0.0)
    y = jnp.dot(a, w_ref[...], preferred_element_type=jnp.float32) + b_ref[...]
    y_ref[...] = y
    p1 = jnp.sum(y, axis=0, keepdims=True)
    p2 = jnp.sum(y * y, axis=0, keepdims=True)

    @pl.when(pl.program_id(0) == 0)
    def _init():
        s1_ref[...] = p1
        s2_ref[...] = p2

    @pl.when(pl.program_id(0) > 0)
    def _acc():
        s1_ref[...] += p1
        s2_ref[...] += p2


def _mmsub_kernel(x_ref, ctr_ref, w_ref, b_ref, y_ref, s1_ref, s2_ref):
    x = x_ref[...]
    ctr_pad = jnp.pad(ctr_ref[...], ((0, 0), (0, x.shape[1] - 3)))
    y = jnp.dot(x - ctr_pad, w_ref[...],
                preferred_element_type=jnp.float32) + b_ref[...]
    y_ref[...] = y
    p1 = jnp.sum(y, axis=0, keepdims=True)
    p2 = jnp.sum(y * y, axis=0, keepdims=True)

    @pl.when(pl.program_id(0) == 0)
    def _init():
        s1_ref[...] = p1
        s2_ref[...] = p2

    @pl.when(pl.program_id(0) > 0)
    def _acc():
        s1_ref[...] += p1
        s2_ref[...] += p2


def _mmsub(x, ctr, w, b, rblk):
    r, cin = x.shape
    cout = w.shape[1]
    wp = jnp.pad(w, ((0, cin - w.shape[0]), (0, 0)))
    out_specs, out_shape = _stats_out(r, cout, rblk)
    return pl.pallas_call(
        _mmsub_kernel,
        grid=(r // rblk,),
        in_specs=[pl.BlockSpec((rblk, cin), lambda i: (i, 0)),
                  pl.BlockSpec((rblk, 3), lambda i: (i, 0)),
                  pl.BlockSpec((cin, cout), lambda i: (0, 0)),
                  pl.BlockSpec((1, cout), lambda i: (0, 0))],
        out_specs=out_specs,
        out_shape=out_shape,
    )(x, ctr, wp, b.reshape(1, cout))


def _stats_out(r, cout, rblk):
    specs = [pl.BlockSpec((rblk, cout), lambda i: (i, 0)),
             pl.BlockSpec((1, cout), lambda i: (0, 0)),
             pl.BlockSpec((1, cout), lambda i: (0, 0))]
    shapes = [jax.ShapeDtypeStruct((r, cout), jnp.float32),
              jax.ShapeDtypeStruct((1, cout), jnp.float32),
              jax.ShapeDtypeStruct((1, cout), jnp.float32)]
    return specs, shapes


def _mm(x, w, b, rblk):
    r, cin = x.shape
    cout = w.shape[1]
    out_specs, out_shape = _stats_out(r, cout, rblk)
    return pl.pallas_call(
        _mm_kernel,
        grid=(r // rblk,),
        in_specs=[pl.BlockSpec((rblk, cin), lambda i: (i, 0)),
                  pl.BlockSpec((cin, cout), lambda i: (0, 0)),
                  pl.BlockSpec((1, cout), lambda i: (0, 0))],
        out_specs=out_specs,
        out_shape=out_shape,
    )(x, w, b.reshape(1, cout))


def _mmbn(x, s1, s2, g, be, w, b, inv_cnt, rblk):
    r, cin = x.shape
    cout = w.shape[1]
    out_specs, out_shape = _stats_out(r, cout, rblk)
    stat_spec = pl.BlockSpec((1, cin), lambda i: (0, 0))
    return pl.pallas_call(
        functools.partial(_mmbn_kernel, inv_cnt=inv_cnt),
        grid=(r // rblk,),
        in_specs=[pl.BlockSpec((rblk, cin), lambda i: (i, 0)),
                  stat_spec, stat_spec, stat_spec, stat_spec,
                  pl.BlockSpec((cin, cout), lambda i: (0, 0)),
                  pl.BlockSpec((1, cout), lambda i: (0, 0))],
        out_specs=out_specs,
        out_shape=out_shape,
    )(x, s1, s2, g.reshape(1, cin), be.reshape(1, cin), w, b.reshape(1, cout))


def _pool_kernel(y_ref, s1_ref, s2_ref, g_ref, be_ref, o_ref, *, inv_cnt):
    d = y_ref.shape[2]
    m = (s1_ref[...] * inv_cnt).reshape(1, 1, d)
    v = (s2_ref[...] * inv_cnt).reshape(1, 1, d) - m * m
    g = g_ref[...].reshape(1, 1, d)
    be = be_ref[...].reshape(1, 1, d)
    a = jnp.maximum(g * (y_ref[...] - m) / jnp.sqrt(v + 1e-5) + be, 0.0)
    o_ref[...] = jnp.max(a, axis=1)


def _pool(y3, s1, s2, g, be, inv_cnt, gb):
    rows, ns, d = y3.shape
    stat_spec = pl.BlockSpec((1, d), lambda i: (0, 0))
    return pl.pallas_call(
        functools.partial(_pool_kernel, inv_cnt=inv_cnt),
        grid=(rows // gb,),
        in_specs=[pl.BlockSpec((gb, ns, d), lambda i: (i, 0, 0)),
                  stat_spec, stat_spec, stat_spec, stat_spec],
        out_specs=pl.BlockSpec((gb, d), lambda i: (i, 0)),
        out_shape=jax.ShapeDtypeStruct((rows, d), jnp.float32),
        compiler_params=pltpu.CompilerParams(
            dimension_semantics=("parallel",)),
    )(y3, s1, s2, g.reshape(1, d), be.reshape(1, d))


def _sa_msg(xs, ys, zs, pts, npoint, specs, params, rb, lw=128):
    b, n = xs.shape
    cx, cy, cz = _fps_centers(xs, ys, zs, npoint)
    (ra, nsa, _), (rbb, nsb, _) = specs
    ia, ib = _group2i(xs, ys, zs, cx, cy, cz, ra, nsa, rbb, nsb, rb, lw)
    c = pts.shape[-1]
    table = jnp.pad(pts.reshape(b * n, c), ((0, 0), (0, 128 - c)))
    groups = [_sc_gather_rows(table, ia, b * npoint * nsa),
              _sc_gather_rows(table, ib, b * npoint * nsb)]
    ctr3 = jnp.stack([cx, cy, cz], axis=-1).reshape(b * npoint, 1, 3)
    outs = []
    for ((radius, ns, dims), mlp, grouped) in zip(specs, params, groups):
        inv_cnt = 1.0 / (b * npoint * ns)
        w, bb, g, be = mlp[0]
        ctr_rows = jnp.broadcast_to(ctr3, (b * npoint, ns, 3)).reshape(
            b * npoint * ns, 3)
        y, s1, s2 = _mmsub(grouped, ctr_rows, w, bb, rblk=8192)
        for w2, b2, g2, be2 in mlp[1:]:
            y, s1n, s2n = _mmbn(y, s1, s2, g, be, w2, b2, inv_cnt, rblk=8192)
            s1, s2, g, be = s1n, s2n, g2, be2
        pooled = _pool(y.reshape(b * npoint, ns, dims[-1]), s1, s2, g, be,
                       inv_cnt, gb=512)
        outs.append(pooled.reshape(b, npoint, dims[-1]))
    return (cx, cy, cz), jnp.concatenate(outs, axis=-1)


def kernel(pointcloud, params):
    b, n, _ = pointcloud.shape
    xs = pointcloud[..., 0]
    ys = pointcloud[..., 1]
    zs = pointcloud[..., 2]
    (cx1, cy1, cz1), f1 = _sa_msg(xs, ys, zs, pointcloud, _NPOINT1,
                                  _SA1_SPECS, params["sa1"], rb=64)
    pts2 = jnp.concatenate([jnp.stack([cx1, cy1, cz1], axis=-1), f1], axis=-1)
    (cx2, cy2, cz2), f2 = _sa_msg(cx1, cy1, cz1, pts2, _NPOINT2,
                                  _SA2_SPECS, params["sa2"], rb=64)
    lin, _, _ = _mm(f2.reshape(b * _NPOINT2, f2.shape[-1]),
                    params["linear_w"], params["linear_b"], rblk=512)
    xyz2 = jnp.stack([cx2, cy2, cz2], axis=-1)
    return jnp.concatenate([xyz2, lin.reshape(b, _NPOINT2, -1)], axis=-1)
